# baseline (speedup 1.0000x reference)
import functools

import jax
import jax.numpy as jnp
from jax import lax
from jax.experimental import pallas as pl
from jax.experimental.pallas import tpu as pltpu
from jax.experimental.pallas import tpu_sc as plsc

_N = 4096
_F = 256
_H = 1024
_NODES = 7
_T = 256
_W = _F + 128
_NT0 = _N // _T
_M1 = _N + _T
_NT1 = _M1 // _T
_M2 = _N + 4 * _T
_NT2 = _M2 // _T
_NPAD = _N + _M2

_HIGH = jax.lax.Precision.HIGHEST


def _mlp_tile(x, w1_ref, b1_ref, w2_ref, b2_ref, w3_ref, b3_ref):
    h = jnp.tanh(jnp.dot(x, w1_ref[0], preferred_element_type=jnp.float32)
                 + b1_ref[0])
    h = jnp.tanh(jnp.dot(h, w2_ref[0], preferred_element_type=jnp.float32)
                 + b2_ref[0])
    logits = (jnp.dot(h, w3_ref[0], preferred_element_type=jnp.float32)
              + b3_ref[0])
    return (logits[:, 0:1] >= logits[:, 1:2]).astype(jnp.float32)


def _tri_ranks(m, U, Ls):
    r = m.shape[0]
    c = jnp.dot(m, U, precision=_HIGH, preferred_element_type=jnp.float32)
    rowtot = c[:, -1:]
    rowoff = jnp.dot(Ls, rowtot, precision=_HIGH,
                     preferred_element_type=jnp.float32)
    rank = c + rowoff - 1.0
    cnt = rowoff[r - 1:r, :] + rowtot[r - 1:r, :]
    return rank, cnt


def _tris(rows):
    ri = lax.broadcasted_iota(jnp.int32, (_T, _T), 0)
    ci = lax.broadcasted_iota(jnp.int32, (_T, _T), 1)
    U = (ri <= ci).astype(jnp.float32)
    ri2 = lax.broadcasted_iota(jnp.int32, (rows, rows), 0)
    ci2 = lax.broadcasted_iota(jnp.int32, (rows, rows), 1)
    Ls = (ri2 > ci2).astype(jnp.float32)
    return U, Ls


def _roundup_t(v):
    return jnp.floor((v + float(_T - 1)) / float(_T)) * float(_T)


_WSPECS = [
    pl.BlockSpec((1, _F, _H), lambda j, nm: (nm[j], 0, 0)),
    pl.BlockSpec((1, 1, _H), lambda j, nm: (nm[j], 0, 0)),
    pl.BlockSpec((1, _H, _H), lambda j, nm: (nm[j], 0, 0)),
    pl.BlockSpec((1, 1, _H), lambda j, nm: (nm[j], 0, 0)),
    pl.BlockSpec((1, _H, 2), lambda j, nm: (nm[j], 0, 0)),
    pl.BlockSpec((1, 1, 2), lambda j, nm: (nm[j], 0, 0)),
]


def _l0_kernel(nm_ref, x_ref, w1_ref, b1_ref, w2_ref, b2_ref, w3_ref,
               b3_ref, dst_ref, meta_ref, nm1_ref, dscr):
    del nm_ref
    j = pl.program_id(0)
    cmp = _mlp_tile(x_ref[:, 0:_F], w1_ref, b1_ref, w2_ref, b2_ref, w3_ref,
                    b3_ref)
    dscr[pl.ds(j, 1)] = cmp[None]

    @pl.when(j == _NT0 - 1)
    def _finish():
        d = dscr[:, :, 0]
        U, Ls = _tris(_NT0)
        rank_l, cnt0 = _tri_ranks(d, U, Ls)
        rank_r, cnt1 = _tri_ranks(1.0 - d, U, Ls)
        s2 = _roundup_t(cnt0)
        dst = d * rank_l + (1.0 - d) * (s2 + rank_r)
        dst_ref[...] = dst.astype(jnp.int32)
        meta_ref[...] = jnp.concatenate(
            [s2, cnt0, cnt1, jnp.zeros((1, 5), jnp.float32)], axis=1)
        jt = (lax.broadcasted_iota(jnp.int32, (1, _NT1), 1)
              .astype(jnp.float32) * float(_T))
        nm1_ref[...] = 1 + (jt >= s2).astype(jnp.int32)


def _run_l0(nm0, xarr, *weights):
    return pl.pallas_call(
        _l0_kernel,
        grid_spec=pltpu.PrefetchScalarGridSpec(
            num_scalar_prefetch=1,
            grid=(_NT0,),
            in_specs=[pl.BlockSpec((_T, _F), lambda j, nm: (j, 0))] + _WSPECS,
            out_specs=[
                pl.BlockSpec((_NT0, _T), lambda j, nm: (0, 0)),
                pl.BlockSpec((1, 8), lambda j, nm: (0, 0)),
                pl.BlockSpec((1, _NT1), lambda j, nm: (0, 0)),
            ],
            scratch_shapes=[pltpu.VMEM((_NT0, _T, 1), jnp.float32)],
        ),
        out_shape=[
            jax.ShapeDtypeStruct((_NT0, _T), jnp.int32),
            jax.ShapeDtypeStruct((1, 8), jnp.float32),
            jax.ShapeDtypeStruct((1, _NT1), jnp.int32),
        ],
        compiler_params=pltpu.CompilerParams(
            dimension_semantics=("arbitrary",)),
    )(nm0, xarr, *weights)


def _l1_kernel(nm_ref, x_ref, w1_ref, b1_ref, w2_ref, b2_ref, w3_ref,
               b3_ref, meta1_ref, dst_ref, meta2_ref, nm2_ref, dscr):
    j = pl.program_id(0)
    cmp = _mlp_tile(x_ref[:, 0:_F], w1_ref, b1_ref, w2_ref, b2_ref, w3_ref,
                    b3_ref)
    dscr[pl.ds(j, 1)] = cmp[None]

    @pl.when(j == _NT1 - 1)
    def _finish():
        d = dscr[:, :, 0]
        s2 = meta1_ref[0, 0]
        c1 = meta1_ref[0, 1]
        c2 = meta1_ref[0, 2]
        U, Ls = _tris(_NT1)
        pos = (lax.broadcasted_iota(jnp.int32, (_NT1, _T), 0) * _T
               + lax.broadcasted_iota(jnp.int32, (_NT1, _T), 1)
               ).astype(jnp.float32)
        pright = pos >= s2
        real = (pos < c1) | (pright & (pos < s2 + c2))
        dst = jnp.zeros_like(d)
        t = jnp.zeros((1, 1), jnp.float32)
        ts, es = [], []
        for b in range(4):
            want_right = (b // 2) == 1
            want_d = (b % 2) == 0
            m = (real & (pright == want_right)
                 & ((d > 0.5) == want_d)).astype(jnp.float32)
            rank, cnt = _tri_ranks(m, U, Ls)
            ts.append(t)
            es.append(t + cnt)
            dst = dst + m * (t + rank)
            t = _roundup_t(t + cnt)
        m_tr = 1.0 - real.astype(jnp.float32)
        rank_tr, _ = _tri_ranks(m_tr, U, Ls)
        dst = dst + m_tr * (t + rank_tr)
        dst_ref[...] = dst.astype(jnp.int32)
        meta2_ref[...] = jnp.concatenate(ts + es, axis=1)
        jt = (lax.broadcasted_iota(jnp.int32, (1, _NT2), 1)
              .astype(jnp.float32) * float(_T))
        nm2_ref[...] = 3 + sum(
            (jt >= ts[g]).astype(jnp.int32) for g in (1, 2, 3))


def _run_l1(nm1, rows1, W1, b1r, W2, b2r, W3, b3r, meta1):
    return pl.pallas_call(
        _l1_kernel,
        grid_spec=pltpu.PrefetchScalarGridSpec(
            num_scalar_prefetch=1,
            grid=(_NT1,),
            in_specs=[pl.BlockSpec((_T, _F), lambda j, nm: (j, 0))]
            + _WSPECS
            + [pl.BlockSpec(memory_space=pltpu.SMEM)],
            out_specs=[
                pl.BlockSpec((_NT1, _T), lambda j, nm: (0, 0)),
                pl.BlockSpec((1, 8), lambda j, nm: (0, 0)),
                pl.BlockSpec((1, _NT2), lambda j, nm: (0, 0)),
            ],
            scratch_shapes=[pltpu.VMEM((_NT1, _T, 1), jnp.float32)],
        ),
        out_shape=[
            jax.ShapeDtypeStruct((_NT1, _T), jnp.int32),
            jax.ShapeDtypeStruct((1, 8), jnp.float32),
            jax.ShapeDtypeStruct((1, _NT2), jnp.int32),
        ],
        compiler_params=pltpu.CompilerParams(
            dimension_semantics=("arbitrary",)),
    )(nm1, rows1, W1, b1r, W2, b2r, W3, b3r, meta1)


def _l2_kernel(nm_ref, xw_ref, w1_ref, b1_ref, w2_ref, b2_ref, w3_ref,
               b3_ref, meta_ref, lb_ref, vals_ref, tgt_ref):
    j = pl.program_id(0)
    dd = _mlp_tile(xw_ref[:, 0:_F], w1_ref, b1_ref, w2_ref, b2_ref, w3_ref,
                   b3_ref)

    node = nm_ref[j]
    e_g = meta_ref[0, 4 + (node - 3)]
    posf = ((j * _T).astype(jnp.float32)
            + lax.broadcasted_iota(jnp.int32, (_T, 1), 0).astype(jnp.float32))
    real = posf < e_g
    idcol = xw_ref[:, _F:_F + 1]
    tgt_ref[...] = jnp.where(real, idcol,
                             jnp.float32(_N) + posf).astype(jnp.int32)[None]

    leaf = 2.0 * node.astype(jnp.float32) + 2.0 - dd - 7.0
    out = jnp.zeros_like(dd)
    for k in range(8):
        out = jnp.where(leaf == float(k), lb_ref[k], out)
    vals_ref[...] = jnp.broadcast_to(out, (_T, 128))[None]


def _run_l2(nm2, rows2, W1, b1r, W2, b2r, W3, b3r, meta2, leaf_best):
    return pl.pallas_call(
        _l2_kernel,
        grid_spec=pltpu.PrefetchScalarGridSpec(
            num_scalar_prefetch=1,
            grid=(_NT2,),
            in_specs=[pl.BlockSpec((_T, _W), lambda j, nm: (j, 0))]
            + _WSPECS
            + [pl.BlockSpec(memory_space=pltpu.SMEM),
               pl.BlockSpec(memory_space=pltpu.SMEM)],
            out_specs=[
                pl.BlockSpec((1, _T, 128), lambda j, nm: (j, 0, 0)),
                pl.BlockSpec((1, _T, 1), lambda j, nm: (j, 0, 0)),
            ],
        ),
        out_shape=[
            jax.ShapeDtypeStruct((_NT2, _T, 128), jnp.float32),
            jax.ShapeDtypeStruct((_NT2, _T, 1), jnp.int32),
        ],
        compiler_params=pltpu.CompilerParams(
            dimension_semantics=("arbitrary",)),
    )(nm2, rows2, W1, b1r, W2, b2r, W3, b3r, meta2, leaf_best)


_NC = 2
_NS = 16
_NWORK = _NC * _NS


def _sc_mesh():
    return plsc.VectorSubcoreMesh(core_axis_name="c", subcore_axis_name="s",
                                  num_cores=_NC, num_subcores=_NS)


def _split_parts(ch):
    parts = []
    off = 0
    while off < ch:
        b = min(128, ch - off)
        assert b % 8 == 0
        parts.append((off, b))
        off += b
    return parts


def _sc_permute(src, dst, m_out):
    m_in = src.shape[0]
    ch = m_in // _NWORK
    parts = _split_parts(ch)
    idx_parts = [
        dst.reshape(_NWORK, ch)[:, off:off + b] for off, b in parts
    ]

    @functools.partial(
        pl.kernel,
        out_type=jax.ShapeDtypeStruct((m_out, _W), jnp.float32),
        mesh=_sc_mesh(),
        scratch_types=(
            [pltpu.VMEM((b,), jnp.int32) for _, b in parts]
            + [pltpu.VMEM((ch, _W), jnp.float32), pltpu.SemaphoreType.DMA]
        ),
    )
    def k(src_hbm, *rest):
        idx_hbms = rest[:len(parts)]
        out_hbm = rest[len(parts)]
        idx_vs = rest[len(parts) + 1:2 * len(parts) + 1]
        rows_v, sem = rest[2 * len(parts) + 1], rest[2 * len(parts) + 2]
        wid = lax.axis_index("s") * _NC + lax.axis_index("c")
        base = wid * ch
        pltpu.sync_copy(src_hbm.at[pl.ds(base, ch)], rows_v)
        for ih, iv in zip(idx_hbms, idx_vs):
            pltpu.sync_copy(ih.at[wid], iv)
        for (off, b), iv in zip(parts, idx_vs):
            pltpu.async_copy(rows_v.at[pl.ds(off, b)],
                             out_hbm.at[iv], sem).wait()

    return k(src, *idx_parts)


def _sc_scatter_out(vals, tgt):
    m = vals.shape[0]
    ch = m // _NWORK
    parts = _split_parts(ch)
    idx_parts = [
        tgt.reshape(_NWORK, ch)[:, off:off + b] for off, b in parts
    ]

    @functools.partial(
        pl.kernel,
        out_type=jax.ShapeDtypeStruct((_NPAD, 128), jnp.float32),
        mesh=_sc_mesh(),
        scratch_types=(
            [pltpu.VMEM((b,), jnp.int32) for _, b in parts]
            + [pltpu.VMEM((ch, 128), jnp.float32), pltpu.SemaphoreType.DMA]
        ),
    )
    def k(vals_hbm, *rest):
        idx_hbms = rest[:len(parts)]
        out_hbm = rest[len(parts)]
        idx_vs = rest[len(parts) + 1:2 * len(parts) + 1]
        vals_v, sem = rest[2 * len(parts) + 1], rest[2 * len(parts) + 2]
        wid = lax.axis_index("s") * _NC + lax.axis_index("c")
        base = wid * ch
        pltpu.sync_copy(vals_hbm.at[pl.ds(base, ch)], vals_v)
        for ih, iv in zip(idx_hbms, idx_vs):
            pltpu.sync_copy(ih.at[wid], iv)
        for (off, b), iv in zip(parts, idx_vs):
            pltpu.async_copy(vals_v.at[pl.ds(off, b)],
                             out_hbm.at[iv], sem).wait()

    return k(vals, *idx_parts)


def kernel(x, W1, b1, W2, b2, W3, b3, leaf_best):
    assert x.shape == (_N, _F) and W1.shape == (_NODES, _F, _H)
    b1r = b1[:, None, :]
    b2r = b2[:, None, :]
    b3r = b3[:, None, :]
    weights = (W1, b1r, W2, b2r, W3, b3r)

    ids = jnp.arange(_N, dtype=jnp.float32)[:, None]
    xext = jnp.concatenate(
        [x, ids, jnp.zeros((_N, _W - _F - 1), jnp.float32)], axis=1)

    nm0 = jnp.zeros((_NT0,), jnp.int32)
    dst1, meta1, nm1 = _run_l0(nm0, xext, *weights)
    rows1 = _sc_permute(xext, dst1.reshape(_N), _M1)

    dst2, meta2, nm2 = _run_l1(nm1.reshape(_NT1), rows1, *weights, meta1)
    rows2 = _sc_permute(rows1, dst2.reshape(_M1), _M2)

    vals, tgt = _run_l2(nm2.reshape(_NT2), rows2, *weights, meta2, leaf_best)
    out_pad = _sc_scatter_out(vals.reshape(_M2, 128), tgt.reshape(_M2))
    return out_pad[:_N, 0]


# speedup vs baseline: 1.0073x; 1.0073x over previous
import functools

import jax
import jax.numpy as jnp
from jax import lax
from jax.experimental import pallas as pl
from jax.experimental.pallas import tpu as pltpu
from jax.experimental.pallas import tpu_sc as plsc

_N = 4096
_F = 256
_H = 1024
_NODES = 7
_T = 256
_T0 = 512
_NT0 = _N // _T0
_M1 = _N + _T
_NT1 = _M1 // _T
_M2 = _N + 4 * _T
_NT2 = _M2 // _T
_NPAD = _N + _M2

_HIGH = jax.lax.Precision.HIGHEST


def _mlp_tile(x, w1_ref, b1_ref, w2_ref, b2_ref, w3_ref, b3_ref):
    h = jnp.tanh(jnp.dot(x, w1_ref[0], preferred_element_type=jnp.float32)
                 + b1_ref[0])
    h = jnp.tanh(jnp.dot(h, w2_ref[0], preferred_element_type=jnp.float32)
                 + b2_ref[0])
    logits = (jnp.dot(h, w3_ref[0], preferred_element_type=jnp.float32)
              + b3_ref[0])
    return (logits[:, 0:1] >= logits[:, 1:2]).astype(jnp.float32)


def _tri_ranks(m, U, Ls):
    r = m.shape[0]
    c = jnp.dot(m, U, precision=_HIGH, preferred_element_type=jnp.float32)
    rowtot = c[:, -1:]
    rowoff = jnp.dot(Ls, rowtot, precision=_HIGH,
                     preferred_element_type=jnp.float32)
    rank = c + rowoff - 1.0
    cnt = rowoff[r - 1:r, :] + rowtot[r - 1:r, :]
    return rank, cnt


def _tris(rows, cols):
    ri = lax.broadcasted_iota(jnp.int32, (cols, cols), 0)
    ci = lax.broadcasted_iota(jnp.int32, (cols, cols), 1)
    U = (ri <= ci).astype(jnp.float32)
    ri2 = lax.broadcasted_iota(jnp.int32, (rows, rows), 0)
    ci2 = lax.broadcasted_iota(jnp.int32, (rows, rows), 1)
    Ls = (ri2 > ci2).astype(jnp.float32)
    return U, Ls


def _roundup_t(v):
    return jnp.floor((v + float(_T - 1)) / float(_T)) * float(_T)


def _wspecs(idx_fn):
    return [
        pl.BlockSpec((1, _F, _H), lambda j, nm: (idx_fn(j, nm), 0, 0)),
        pl.BlockSpec((1, 1, _H), lambda j, nm: (idx_fn(j, nm), 0, 0)),
        pl.BlockSpec((1, _H, _H), lambda j, nm: (idx_fn(j, nm), 0, 0)),
        pl.BlockSpec((1, 1, _H), lambda j, nm: (idx_fn(j, nm), 0, 0)),
        pl.BlockSpec((1, _H, 2), lambda j, nm: (idx_fn(j, nm), 0, 0)),
        pl.BlockSpec((1, 1, 2), lambda j, nm: (idx_fn(j, nm), 0, 0)),
    ]


def _l0_kernel(nm_ref, x_ref, w1_ref, b1_ref, w2_ref, b2_ref, w3_ref,
               b3_ref, dst_ref, meta_ref, nm1_ref, dscr):
    del nm_ref
    j = pl.program_id(0)
    cmp = _mlp_tile(x_ref[...], w1_ref, b1_ref, w2_ref, b2_ref, w3_ref,
                    b3_ref)
    dscr[pl.ds(j, 1)] = cmp[None]

    @pl.when(j == _NT0 - 1)
    def _finish():
        d = dscr[:, :, 0]
        U, Ls = _tris(_NT0, _T0)
        rank_l, cnt0 = _tri_ranks(d, U, Ls)
        rank_r, cnt1 = _tri_ranks(1.0 - d, U, Ls)
        s2 = _roundup_t(cnt0)
        dst = d * rank_l + (1.0 - d) * (s2 + rank_r)
        dst_ref[...] = dst.astype(jnp.int32)
        meta_ref[...] = jnp.concatenate(
            [s2, cnt0, cnt1, jnp.zeros((1, 5), jnp.float32)], axis=1)
        jt = (lax.broadcasted_iota(jnp.int32, (1, _NT1), 1)
              .astype(jnp.float32) * float(_T))
        nm1_ref[...] = 1 + (jt >= s2).astype(jnp.int32)


def _run_l0(xarr, *weights):
    nm0 = jnp.zeros((1,), jnp.int32)
    return pl.pallas_call(
        _l0_kernel,
        grid_spec=pltpu.PrefetchScalarGridSpec(
            num_scalar_prefetch=1,
            grid=(_NT0,),
            in_specs=[pl.BlockSpec((_T0, _F), lambda j, nm: (j, 0))]
            + _wspecs(lambda j, nm: 0),
            out_specs=[
                pl.BlockSpec((_NT0, _T0), lambda j, nm: (0, 0)),
                pl.BlockSpec((1, 8), lambda j, nm: (0, 0)),
                pl.BlockSpec((1, _NT1), lambda j, nm: (0, 0)),
            ],
            scratch_shapes=[pltpu.VMEM((_NT0, _T0, 1), jnp.float32)],
        ),
        out_shape=[
            jax.ShapeDtypeStruct((_NT0, _T0), jnp.int32),
            jax.ShapeDtypeStruct((1, 8), jnp.float32),
            jax.ShapeDtypeStruct((1, _NT1), jnp.int32),
        ],
        compiler_params=pltpu.CompilerParams(
            dimension_semantics=("arbitrary",)),
    )(nm0, xarr, *weights)


def _l1_kernel(nm_ref, x_ref, w1_ref, b1_ref, w2_ref, b2_ref, w3_ref,
               b3_ref, meta1_ref, dst_ref, meta2_ref, nm2_ref, dscr):
    j = pl.program_id(0)
    cmp = _mlp_tile(x_ref[...], w1_ref, b1_ref, w2_ref, b2_ref, w3_ref,
                    b3_ref)
    dscr[pl.ds(j, 1)] = cmp[None]

    @pl.when(j == _NT1 - 1)
    def _finish():
        d = dscr[:, :, 0]
        s2 = meta1_ref[0, 0]
        c1 = meta1_ref[0, 1]
        c2 = meta1_ref[0, 2]
        U, Ls = _tris(_NT1, _T)
        pos = (lax.broadcasted_iota(jnp.int32, (_NT1, _T), 0) * _T
               + lax.broadcasted_iota(jnp.int32, (_NT1, _T), 1)
               ).astype(jnp.float32)
        pright = pos >= s2
        real = (pos < c1) | (pright & (pos < s2 + c2))
        dst = jnp.zeros_like(d)
        t = jnp.zeros((1, 1), jnp.float32)
        ts, es = [], []
        for b in range(4):
            want_right = (b // 2) == 1
            want_d = (b % 2) == 0
            m = (real & (pright == want_right)
                 & ((d > 0.5) == want_d)).astype(jnp.float32)
            rank, cnt = _tri_ranks(m, U, Ls)
            ts.append(t)
            es.append(t + cnt)
            dst = dst + m * (t + rank)
            t = _roundup_t(t + cnt)
        m_tr = 1.0 - real.astype(jnp.float32)
        rank_tr, _ = _tri_ranks(m_tr, U, Ls)
        dst = dst + m_tr * (t + rank_tr)
        dst_ref[...] = dst.astype(jnp.int32)
        meta2_ref[...] = jnp.concatenate(ts + es, axis=1)
        jt = (lax.broadcasted_iota(jnp.int32, (1, _NT2), 1)
              .astype(jnp.float32) * float(_T))
        nm2_ref[...] = 3 + sum(
            (jt >= ts[g]).astype(jnp.int32) for g in (1, 2, 3))


def _run_l1(nm1, rows1, W1, b1r, W2, b2r, W3, b3r, meta1):
    return pl.pallas_call(
        _l1_kernel,
        grid_spec=pltpu.PrefetchScalarGridSpec(
            num_scalar_prefetch=1,
            grid=(_NT1,),
            in_specs=[pl.BlockSpec((_T, _F), lambda j, nm: (j, 0))]
            + _wspecs(lambda j, nm: nm[j])
            + [pl.BlockSpec(memory_space=pltpu.SMEM)],
            out_specs=[
                pl.BlockSpec((_NT1, _T), lambda j, nm: (0, 0)),
                pl.BlockSpec((1, 8), lambda j, nm: (0, 0)),
                pl.BlockSpec((1, _NT2), lambda j, nm: (0, 0)),
            ],
            scratch_shapes=[pltpu.VMEM((_NT1, _T, 1), jnp.float32)],
        ),
        out_shape=[
            jax.ShapeDtypeStruct((_NT1, _T), jnp.int32),
            jax.ShapeDtypeStruct((1, 8), jnp.float32),
            jax.ShapeDtypeStruct((1, _NT2), jnp.int32),
        ],
        compiler_params=pltpu.CompilerParams(
            dimension_semantics=("arbitrary",)),
    )(nm1, rows1, W1, b1r, W2, b2r, W3, b3r, meta1)


def _l2_kernel(nm_ref, x_ref, ids_ref, w1_ref, b1_ref, w2_ref, b2_ref,
               w3_ref, b3_ref, meta_ref, lb_ref, vals_ref, tgt_ref):
    j = pl.program_id(0)
    dd = _mlp_tile(x_ref[...], w1_ref, b1_ref, w2_ref, b2_ref, w3_ref,
                   b3_ref)
    node = nm_ref[j]
    e_g = meta_ref[0, 4 + (node - 3)]
    posi = (j * _T) + lax.broadcasted_iota(jnp.int32, (_T, 1), 0)
    real = posi.astype(jnp.float32) < e_g
    tgt_ref[...] = jnp.where(real, ids_ref[:, 0:1], _N + posi)[None]

    leaf = 2.0 * node.astype(jnp.float32) + 2.0 - dd - 7.0
    out = jnp.zeros_like(dd)
    for k in range(8):
        out = jnp.where(leaf == float(k), lb_ref[k], out)
    vals_ref[...] = jnp.broadcast_to(out, (_T, 128))[None]


def _run_l2(nm2, rows2, ids2, W1, b1r, W2, b2r, W3, b3r, meta2, leaf_best):
    return pl.pallas_call(
        _l2_kernel,
        grid_spec=pltpu.PrefetchScalarGridSpec(
            num_scalar_prefetch=1,
            grid=(_NT2,),
            in_specs=[pl.BlockSpec((_T, _F), lambda j, nm: (j, 0)),
                      pl.BlockSpec((_T, 128), lambda j, nm: (j, 0))]
            + _wspecs(lambda j, nm: nm[j])
            + [pl.BlockSpec(memory_space=pltpu.SMEM),
               pl.BlockSpec(memory_space=pltpu.SMEM)],
            out_specs=[
                pl.BlockSpec((1, _T, 128), lambda j, nm: (j, 0, 0)),
                pl.BlockSpec((1, _T, 1), lambda j, nm: (j, 0, 0)),
            ],
        ),
        out_shape=[
            jax.ShapeDtypeStruct((_NT2, _T, 128), jnp.float32),
            jax.ShapeDtypeStruct((_NT2, _T, 1), jnp.int32),
        ],
        compiler_params=pltpu.CompilerParams(
            dimension_semantics=("arbitrary",)),
    )(nm2, rows2, ids2, W1, b1r, W2, b2r, W3, b3r, meta2, leaf_best)


_NC = 2
_NS = 16
_NWORK = _NC * _NS
_CH = 128


def _sc_mesh():
    return plsc.VectorSubcoreMesh(core_axis_name="c", subcore_axis_name="s",
                                  num_cores=_NC, num_subcores=_NS)


def _sc_permute(src, ids, dst, m_out):
    m_in = src.shape[0]
    nch = m_in // _CH
    nloop = (nch + _NWORK - 1) // _NWORK
    idx = dst.reshape(nch, _CH)

    @functools.partial(
        pl.kernel,
        out_type=(jax.ShapeDtypeStruct((m_out, _F), jnp.float32),
                  jax.ShapeDtypeStruct((m_out, 128), jnp.int32)),
        mesh=_sc_mesh(),
        scratch_types=[
            pltpu.VMEM((_CH,), jnp.int32),
            pltpu.VMEM((_CH, _F), jnp.float32),
            pltpu.VMEM((_CH, 128), jnp.int32),
            pltpu.SemaphoreType.DMA,
        ],
    )
    def k(src_hbm, ids_hbm, idx_hbm, out_hbm, ido_hbm, idx_v, rows_v,
          ids_v, sem):
        wid = lax.axis_index("s") * _NC + lax.axis_index("c")

        def chunk(cid):
            pltpu.sync_copy(src_hbm.at[pl.ds(cid * _CH, _CH)], rows_v)
            pltpu.sync_copy(ids_hbm.at[pl.ds(cid * _CH, _CH)], ids_v)
            pltpu.sync_copy(idx_hbm.at[cid], idx_v)
            pltpu.async_copy(rows_v, out_hbm.at[idx_v], sem).wait()
            pltpu.async_copy(ids_v, ido_hbm.at[idx_v], sem).wait()

        for t in range(nloop):
            cid = wid + _NWORK * t
            if (t + 1) * _NWORK <= nch:
                chunk(cid)
            else:
                @pl.when(cid < nch)
                def _():
                    chunk(cid)

    return k(src, ids, idx)


def _sc_scatter_out(vals, tgt):
    m = vals.shape[0]
    nch = m // _CH
    nloop = (nch + _NWORK - 1) // _NWORK
    idx = tgt.reshape(nch, _CH)

    @functools.partial(
        pl.kernel,
        out_type=jax.ShapeDtypeStruct((_NPAD, 128), jnp.float32),
        mesh=_sc_mesh(),
        scratch_types=[
            pltpu.VMEM((_CH,), jnp.int32),
            pltpu.VMEM((_CH, 128), jnp.float32),
            pltpu.SemaphoreType.DMA,
        ],
    )
    def k(vals_hbm, idx_hbm, out_hbm, idx_v, vals_v, sem):
        wid = lax.axis_index("s") * _NC + lax.axis_index("c")

        def chunk(cid):
            pltpu.sync_copy(vals_hbm.at[pl.ds(cid * _CH, _CH)], vals_v)
            pltpu.sync_copy(idx_hbm.at[cid], idx_v)
            pltpu.async_copy(vals_v, out_hbm.at[idx_v], sem).wait()

        for t in range(nloop):
            cid = wid + _NWORK * t
            if (t + 1) * _NWORK <= nch:
                chunk(cid)
            else:
                @pl.when(cid < nch)
                def _():
                    chunk(cid)

    return k(vals, idx)


def kernel(x, W1, b1, W2, b2, W3, b3, leaf_best):
    assert x.shape == (_N, _F) and W1.shape == (_NODES, _F, _H)
    b1r = b1[:, None, :]
    b2r = b2[:, None, :]
    b3r = b3[:, None, :]
    weights = (W1, b1r, W2, b2r, W3, b3r)

    ids0 = jnp.broadcast_to(jnp.arange(_N, dtype=jnp.int32)[:, None],
                            (_N, 128))

    dst1, meta1, nm1 = _run_l0(x, *weights)
    rows1, ids1 = _sc_permute(x, ids0, dst1.reshape(_N), _M1)

    dst2, meta2, nm2 = _run_l1(nm1.reshape(_NT1), rows1, *weights, meta1)
    rows2, ids2 = _sc_permute(rows1, ids1, dst2.reshape(_M1), _M2)

    vals, tgt = _run_l2(nm2.reshape(_NT2), rows2, ids2, *weights, meta2,
                        leaf_best)
    out_pad = _sc_scatter_out(vals.reshape(_M2, 128), tgt.reshape(_M2))
    return out_pad[:_N, 0]


# speedup vs baseline: 1.0503x; 1.0427x over previous
import functools

import jax
import jax.numpy as jnp
from jax import lax
from jax.experimental import pallas as pl
from jax.experimental.pallas import tpu as pltpu
from jax.experimental.pallas import tpu_sc as plsc

_N = 4096
_F = 256
_H = 1024
_NODES = 7
_T = 512
_T0 = 512
_NT0 = _N // _T0
_M1 = _N + _T
_NT1 = _M1 // _T
_M2 = _N + 4 * _T
_NT2 = _M2 // _T
_NPAD = _N + _M2

_HIGH = jax.lax.Precision.HIGHEST


def _mlp_tile(x, w1_ref, b1_ref, w2_ref, b2_ref, w3_ref, b3_ref):
    h = jnp.tanh(jnp.dot(x, w1_ref[0], preferred_element_type=jnp.float32)
                 + b1_ref[0])
    h = jnp.tanh(jnp.dot(h, w2_ref[0], preferred_element_type=jnp.float32)
                 + b2_ref[0])
    logits = (jnp.dot(h, w3_ref[0], preferred_element_type=jnp.float32)
              + b3_ref[0])
    return (logits[:, 0:1] >= logits[:, 1:2]).astype(jnp.float32)


def _tri_ranks(m, U, Ls):
    r = m.shape[0]
    c = jnp.dot(m, U, precision=_HIGH, preferred_element_type=jnp.float32)
    rowtot = c[:, -1:]
    rowoff = jnp.dot(Ls, rowtot, precision=_HIGH,
                     preferred_element_type=jnp.float32)
    rank = c + rowoff - 1.0
    cnt = rowoff[r - 1:r, :] + rowtot[r - 1:r, :]
    return rank, cnt


def _tris(rows, cols):
    ri = lax.broadcasted_iota(jnp.int32, (cols, cols), 0)
    ci = lax.broadcasted_iota(jnp.int32, (cols, cols), 1)
    U = (ri <= ci).astype(jnp.float32)
    ri2 = lax.broadcasted_iota(jnp.int32, (rows, rows), 0)
    ci2 = lax.broadcasted_iota(jnp.int32, (rows, rows), 1)
    Ls = (ri2 > ci2).astype(jnp.float32)
    return U, Ls


def _roundup_t(v):
    return jnp.floor((v + float(_T - 1)) / float(_T)) * float(_T)


def _wspecs(idx_fn):
    return [
        pl.BlockSpec((1, _F, _H), lambda j, nm: (idx_fn(j, nm), 0, 0)),
        pl.BlockSpec((1, 1, _H), lambda j, nm: (idx_fn(j, nm), 0, 0)),
        pl.BlockSpec((1, _H, _H), lambda j, nm: (idx_fn(j, nm), 0, 0)),
        pl.BlockSpec((1, 1, _H), lambda j, nm: (idx_fn(j, nm), 0, 0)),
        pl.BlockSpec((1, _H, 2), lambda j, nm: (idx_fn(j, nm), 0, 0)),
        pl.BlockSpec((1, 1, 2), lambda j, nm: (idx_fn(j, nm), 0, 0)),
    ]


def _l0_kernel(nm_ref, x_ref, w1_ref, b1_ref, w2_ref, b2_ref, w3_ref,
               b3_ref, dst_ref, meta_ref, nm1_ref, dscr):
    del nm_ref
    j = pl.program_id(0)
    cmp = _mlp_tile(x_ref[...], w1_ref, b1_ref, w2_ref, b2_ref, w3_ref,
                    b3_ref)
    dscr[pl.ds(j, 1)] = cmp[None]

    @pl.when(j == _NT0 - 1)
    def _finish():
        d = dscr[:, :, 0]
        U, Ls = _tris(_NT0, _T0)
        rank_l, cnt0 = _tri_ranks(d, U, Ls)
        rank_r, cnt1 = _tri_ranks(1.0 - d, U, Ls)
        s2 = _roundup_t(cnt0)
        dst = d * rank_l + (1.0 - d) * (s2 + rank_r)
        dst_ref[...] = dst.astype(jnp.int32)
        meta_ref[...] = jnp.concatenate(
            [s2, cnt0, cnt1, jnp.zeros((1, 5), jnp.float32)], axis=1)
        jt = (lax.broadcasted_iota(jnp.int32, (1, _NT1), 1)
              .astype(jnp.float32) * float(_T))
        nm1_ref[...] = 1 + (jt >= s2).astype(jnp.int32)


def _run_l0(xarr, *weights):
    nm0 = jnp.zeros((1,), jnp.int32)
    return pl.pallas_call(
        _l0_kernel,
        grid_spec=pltpu.PrefetchScalarGridSpec(
            num_scalar_prefetch=1,
            grid=(_NT0,),
            in_specs=[pl.BlockSpec((_T0, _F), lambda j, nm: (j, 0))]
            + _wspecs(lambda j, nm: 0),
            out_specs=[
                pl.BlockSpec((_NT0, _T0), lambda j, nm: (0, 0)),
                pl.BlockSpec((1, 8), lambda j, nm: (0, 0)),
                pl.BlockSpec((1, _NT1), lambda j, nm: (0, 0)),
            ],
            scratch_shapes=[pltpu.VMEM((_NT0, _T0, 1), jnp.float32)],
        ),
        out_shape=[
            jax.ShapeDtypeStruct((_NT0, _T0), jnp.int32),
            jax.ShapeDtypeStruct((1, 8), jnp.float32),
            jax.ShapeDtypeStruct((1, _NT1), jnp.int32),
        ],
        compiler_params=pltpu.CompilerParams(
            dimension_semantics=("arbitrary",)),
    )(nm0, xarr, *weights)


def _l1_kernel(nm_ref, x_ref, w1_ref, b1_ref, w2_ref, b2_ref, w3_ref,
               b3_ref, meta1_ref, dst_ref, meta2_ref, nm2_ref, dscr):
    j = pl.program_id(0)
    cmp = _mlp_tile(x_ref[...], w1_ref, b1_ref, w2_ref, b2_ref, w3_ref,
                    b3_ref)
    dscr[pl.ds(j, 1)] = cmp[None]

    @pl.when(j == _NT1 - 1)
    def _finish():
        d = dscr[:, :, 0]
        s2 = meta1_ref[0, 0]
        c1 = meta1_ref[0, 1]
        c2 = meta1_ref[0, 2]
        U, Ls = _tris(_NT1, _T)
        pos = (lax.broadcasted_iota(jnp.int32, (_NT1, _T), 0) * _T
               + lax.broadcasted_iota(jnp.int32, (_NT1, _T), 1)
               ).astype(jnp.float32)
        pright = pos >= s2
        real = (pos < c1) | (pright & (pos < s2 + c2))
        dst = jnp.zeros_like(d)
        t = jnp.zeros((1, 1), jnp.float32)
        ts, es = [], []
        for b in range(4):
            want_right = (b // 2) == 1
            want_d = (b % 2) == 0
            m = (real & (pright == want_right)
                 & ((d > 0.5) == want_d)).astype(jnp.float32)
            rank, cnt = _tri_ranks(m, U, Ls)
            ts.append(t)
            es.append(t + cnt)
            dst = dst + m * (t + rank)
            t = _roundup_t(t + cnt)
        m_tr = 1.0 - real.astype(jnp.float32)
        rank_tr, _ = _tri_ranks(m_tr, U, Ls)
        dst = dst + m_tr * (t + rank_tr)
        dst_ref[...] = dst.astype(jnp.int32)
        meta2_ref[...] = jnp.concatenate(ts + es, axis=1)
        jt = (lax.broadcasted_iota(jnp.int32, (1, _NT2), 1)
              .astype(jnp.float32) * float(_T))
        nm2_ref[...] = 3 + sum(
            (jt >= ts[g]).astype(jnp.int32) for g in (1, 2, 3))


def _run_l1(nm1, rows1, W1, b1r, W2, b2r, W3, b3r, meta1):
    return pl.pallas_call(
        _l1_kernel,
        grid_spec=pltpu.PrefetchScalarGridSpec(
            num_scalar_prefetch=1,
            grid=(_NT1,),
            in_specs=[pl.BlockSpec((_T, _F), lambda j, nm: (j, 0))]
            + _wspecs(lambda j, nm: nm[j])
            + [pl.BlockSpec(memory_space=pltpu.SMEM)],
            out_specs=[
                pl.BlockSpec((_NT1, _T), lambda j, nm: (0, 0)),
                pl.BlockSpec((1, 8), lambda j, nm: (0, 0)),
                pl.BlockSpec((1, _NT2), lambda j, nm: (0, 0)),
            ],
            scratch_shapes=[pltpu.VMEM((_NT1, _T, 1), jnp.float32)],
        ),
        out_shape=[
            jax.ShapeDtypeStruct((_NT1, _T), jnp.int32),
            jax.ShapeDtypeStruct((1, 8), jnp.float32),
            jax.ShapeDtypeStruct((1, _NT2), jnp.int32),
        ],
        compiler_params=pltpu.CompilerParams(
            dimension_semantics=("arbitrary",)),
    )(nm1, rows1, W1, b1r, W2, b2r, W3, b3r, meta1)


def _l2_kernel(nm_ref, x_ref, ids_ref, w1_ref, b1_ref, w2_ref, b2_ref,
               w3_ref, b3_ref, meta_ref, lb_ref, vals_ref, tgt_ref):
    j = pl.program_id(0)
    dd = _mlp_tile(x_ref[...], w1_ref, b1_ref, w2_ref, b2_ref, w3_ref,
                   b3_ref)
    node = nm_ref[j]
    e_g = meta_ref[0, 4 + (node - 3)]
    posi = (j * _T) + lax.broadcasted_iota(jnp.int32, (_T, 1), 0)
    real = posi.astype(jnp.float32) < e_g
    tgt_ref[...] = jnp.where(real, ids_ref[:, 0:1], _N + posi)[None]

    leaf = 2.0 * node.astype(jnp.float32) + 2.0 - dd - 7.0
    out = jnp.zeros_like(dd)
    for k in range(8):
        out = jnp.where(leaf == float(k), lb_ref[k], out)
    vals_ref[...] = jnp.broadcast_to(out, (_T, 128))[None]


def _run_l2(nm2, rows2, ids2, W1, b1r, W2, b2r, W3, b3r, meta2, leaf_best):
    return pl.pallas_call(
        _l2_kernel,
        grid_spec=pltpu.PrefetchScalarGridSpec(
            num_scalar_prefetch=1,
            grid=(_NT2,),
            in_specs=[pl.BlockSpec((_T, _F), lambda j, nm: (j, 0)),
                      pl.BlockSpec((_T, 128), lambda j, nm: (j, 0))]
            + _wspecs(lambda j, nm: nm[j])
            + [pl.BlockSpec(memory_space=pltpu.SMEM),
               pl.BlockSpec(memory_space=pltpu.SMEM)],
            out_specs=[
                pl.BlockSpec((1, _T, 128), lambda j, nm: (j, 0, 0)),
                pl.BlockSpec((1, _T, 1), lambda j, nm: (j, 0, 0)),
            ],
        ),
        out_shape=[
            jax.ShapeDtypeStruct((_NT2, _T, 128), jnp.float32),
            jax.ShapeDtypeStruct((_NT2, _T, 1), jnp.int32),
        ],
        compiler_params=pltpu.CompilerParams(
            dimension_semantics=("arbitrary",)),
    )(nm2, rows2, ids2, W1, b1r, W2, b2r, W3, b3r, meta2, leaf_best)


_NC = 2
_NS = 16
_NWORK = _NC * _NS
_CH = 128


def _sc_mesh():
    return plsc.VectorSubcoreMesh(core_axis_name="c", subcore_axis_name="s",
                                  num_cores=_NC, num_subcores=_NS)


def _sc_permute(src, ids, dst, m_out):
    m_in = src.shape[0]
    nch = m_in // _CH
    nloop = (nch + _NWORK - 1) // _NWORK
    idx = dst.reshape(nch, _CH)

    @functools.partial(
        pl.kernel,
        out_type=(jax.ShapeDtypeStruct((m_out, _F), jnp.float32),
                  jax.ShapeDtypeStruct((m_out, 128), jnp.int32)),
        mesh=_sc_mesh(),
        scratch_types=[
            pltpu.VMEM((_CH,), jnp.int32),
            pltpu.VMEM((_CH, _F), jnp.float32),
            pltpu.VMEM((_CH, 128), jnp.int32),
            pltpu.SemaphoreType.DMA,
        ],
    )
    def k(src_hbm, ids_hbm, idx_hbm, out_hbm, ido_hbm, idx_v, rows_v,
          ids_v, sem):
        wid = lax.axis_index("s") * _NC + lax.axis_index("c")

        def chunk(cid):
            pltpu.sync_copy(src_hbm.at[pl.ds(cid * _CH, _CH)], rows_v)
            pltpu.sync_copy(ids_hbm.at[pl.ds(cid * _CH, _CH)], ids_v)
            pltpu.sync_copy(idx_hbm.at[cid], idx_v)
            c1 = pltpu.async_copy(rows_v, out_hbm.at[idx_v], sem)
            c2 = pltpu.async_copy(ids_v, ido_hbm.at[idx_v], sem)
            c1.wait()
            c2.wait()

        for t in range(nloop):
            cid = wid + _NWORK * t
            if (t + 1) * _NWORK <= nch:
                chunk(cid)
            else:
                @pl.when(cid < nch)
                def _():
                    chunk(cid)

    return k(src, ids, idx)


def _sc_scatter_out(vals, tgt):
    m = vals.shape[0]
    nch = m // _CH
    nloop = (nch + _NWORK - 1) // _NWORK
    idx = tgt.reshape(nch, _CH)

    @functools.partial(
        pl.kernel,
        out_type=jax.ShapeDtypeStruct((_NPAD, 128), jnp.float32),
        mesh=_sc_mesh(),
        scratch_types=[
            pltpu.VMEM((_CH,), jnp.int32),
            pltpu.VMEM((_CH, 128), jnp.float32),
            pltpu.SemaphoreType.DMA,
        ],
    )
    def k(vals_hbm, idx_hbm, out_hbm, idx_v, vals_v, sem):
        wid = lax.axis_index("s") * _NC + lax.axis_index("c")

        def chunk(cid):
            pltpu.sync_copy(vals_hbm.at[pl.ds(cid * _CH, _CH)], vals_v)
            pltpu.sync_copy(idx_hbm.at[cid], idx_v)
            pltpu.async_copy(vals_v, out_hbm.at[idx_v], sem).wait()

        for t in range(nloop):
            cid = wid + _NWORK * t
            if (t + 1) * _NWORK <= nch:
                chunk(cid)
            else:
                @pl.when(cid < nch)
                def _():
                    chunk(cid)

    return k(vals, idx)


def kernel(x, W1, b1, W2, b2, W3, b3, leaf_best):
    assert x.shape == (_N, _F) and W1.shape == (_NODES, _F, _H)
    b1r = b1[:, None, :]
    b2r = b2[:, None, :]
    b3r = b3[:, None, :]
    weights = (W1, b1r, W2, b2r, W3, b3r)

    ids0 = jnp.broadcast_to(jnp.arange(_N, dtype=jnp.int32)[:, None],
                            (_N, 128))

    dst1, meta1, nm1 = _run_l0(x, *weights)
    rows1, ids1 = _sc_permute(x, ids0, dst1.reshape(_N), _M1)

    dst2, meta2, nm2 = _run_l1(nm1.reshape(_NT1), rows1, *weights, meta1)
    rows2, ids2 = _sc_permute(rows1, ids1, dst2.reshape(_M1), _M2)

    vals, tgt = _run_l2(nm2.reshape(_NT2), rows2, ids2, *weights, meta2,
                        leaf_best)
    out_pad = _sc_scatter_out(vals.reshape(_M2, 128), tgt.reshape(_M2))
    return out_pad[:_N, 0]


# speedup vs baseline: 1.0744x; 1.0230x over previous
import functools

import jax
import jax.numpy as jnp
from jax import lax
from jax.experimental import pallas as pl
from jax.experimental.pallas import tpu as pltpu
from jax.experimental.pallas import tpu_sc as plsc

_N = 4096
_F = 256
_H = 1024
_NODES = 7
_T = 512
_T0 = 512
_NT0 = _N // _T0
_M1 = _N + _T
_NT1 = _M1 // _T
_M2 = _N + 4 * _T
_NT2 = _M2 // _T
_NPAD = _N + _M2

_HIGH = jax.lax.Precision.HIGHEST


def _mlp_tile(x, w1_ref, b1_ref, w2_ref, b2_ref, w3_ref, b3_ref):
    h = jnp.tanh(jnp.dot(x, w1_ref[0], preferred_element_type=jnp.float32)
                 + b1_ref[0])
    h = jnp.tanh(jnp.dot(h, w2_ref[0], preferred_element_type=jnp.float32)
                 + b2_ref[0])
    logits = (jnp.dot(h, w3_ref[0], preferred_element_type=jnp.float32)
              + b3_ref[0])
    return (logits[:, 0:1] >= logits[:, 1:2]).astype(jnp.float32)


def _tri_ranks(m, U, Ls):
    r = m.shape[0]
    c = jnp.dot(m, U, precision=_HIGH, preferred_element_type=jnp.float32)
    rowtot = c[:, -1:]
    rowoff = jnp.dot(Ls, rowtot, precision=_HIGH,
                     preferred_element_type=jnp.float32)
    rank = c + rowoff - 1.0
    cnt = rowoff[r - 1:r, :] + rowtot[r - 1:r, :]
    return rank, cnt


def _tris(rows, cols):
    ri = lax.broadcasted_iota(jnp.int32, (cols, cols), 0)
    ci = lax.broadcasted_iota(jnp.int32, (cols, cols), 1)
    U = (ri <= ci).astype(jnp.float32)
    ri2 = lax.broadcasted_iota(jnp.int32, (rows, rows), 0)
    ci2 = lax.broadcasted_iota(jnp.int32, (rows, rows), 1)
    Ls = (ri2 > ci2).astype(jnp.float32)
    return U, Ls


def _roundup_t(v):
    return jnp.floor((v + float(_T - 1)) / float(_T)) * float(_T)


def _wspecs(idx_fn):
    return [
        pl.BlockSpec((1, _F, _H), lambda j, nm: (idx_fn(j, nm), 0, 0)),
        pl.BlockSpec((1, 1, _H), lambda j, nm: (idx_fn(j, nm), 0, 0)),
        pl.BlockSpec((1, _H, _H), lambda j, nm: (idx_fn(j, nm), 0, 0)),
        pl.BlockSpec((1, 1, _H), lambda j, nm: (idx_fn(j, nm), 0, 0)),
        pl.BlockSpec((1, _H, 2), lambda j, nm: (idx_fn(j, nm), 0, 0)),
        pl.BlockSpec((1, 1, 2), lambda j, nm: (idx_fn(j, nm), 0, 0)),
    ]


def _l0_kernel(nm_ref, x_ref, w1_ref, b1_ref, w2_ref, b2_ref, w3_ref,
               b3_ref, dst_ref, meta_ref, nm1_ref, dscr):
    del nm_ref
    j = pl.program_id(0)
    cmp = _mlp_tile(x_ref[...], w1_ref, b1_ref, w2_ref, b2_ref, w3_ref,
                    b3_ref)
    dscr[pl.ds(j, 1)] = cmp[None]

    @pl.when(j == _NT0 - 1)
    def _finish():
        d = dscr[:, :, 0]
        U, Ls = _tris(_NT0, _T0)
        rank_l, cnt0 = _tri_ranks(d, U, Ls)
        rank_r, cnt1 = _tri_ranks(1.0 - d, U, Ls)
        s2 = _roundup_t(cnt0)
        dst = d * rank_l + (1.0 - d) * (s2 + rank_r)
        dst_ref[...] = dst.astype(jnp.int32)
        meta_ref[...] = jnp.concatenate(
            [s2, cnt0, cnt1, jnp.zeros((1, 5), jnp.float32)], axis=1)
        jt = (lax.broadcasted_iota(jnp.int32, (1, _NT1), 1)
              .astype(jnp.float32) * float(_T))
        nm1_ref[...] = 1 + (jt >= s2).astype(jnp.int32)


def _run_l0(xarr, *weights):
    nm0 = jnp.zeros((1,), jnp.int32)
    return pl.pallas_call(
        _l0_kernel,
        grid_spec=pltpu.PrefetchScalarGridSpec(
            num_scalar_prefetch=1,
            grid=(_NT0,),
            in_specs=[pl.BlockSpec((_T0, _F), lambda j, nm: (j, 0))]
            + _wspecs(lambda j, nm: 0),
            out_specs=[
                pl.BlockSpec((_NT0, _T0), lambda j, nm: (0, 0)),
                pl.BlockSpec((1, 8), lambda j, nm: (0, 0)),
                pl.BlockSpec((1, _NT1), lambda j, nm: (0, 0)),
            ],
            scratch_shapes=[pltpu.VMEM((_NT0, _T0, 1), jnp.float32)],
        ),
        out_shape=[
            jax.ShapeDtypeStruct((_NT0, _T0), jnp.int32),
            jax.ShapeDtypeStruct((1, 8), jnp.float32),
            jax.ShapeDtypeStruct((1, _NT1), jnp.int32),
        ],
        compiler_params=pltpu.CompilerParams(
            dimension_semantics=("arbitrary",)),
    )(nm0, xarr, *weights)


def _l1_kernel(nm_ref, x_ref, w1_ref, b1_ref, w2_ref, b2_ref, w3_ref,
               b3_ref, meta1_ref, dst_ref, meta2_ref, nm2_ref, dscr):
    j = pl.program_id(0)
    cmp = _mlp_tile(x_ref[...], w1_ref, b1_ref, w2_ref, b2_ref, w3_ref,
                    b3_ref)
    dscr[pl.ds(j, 1)] = cmp[None]

    @pl.when(j == _NT1 - 1)
    def _finish():
        d = dscr[:, :, 0]
        s2 = meta1_ref[0, 0]
        c1 = meta1_ref[0, 1]
        c2 = meta1_ref[0, 2]
        U, Ls = _tris(_NT1, _T)
        pos = (lax.broadcasted_iota(jnp.int32, (_NT1, _T), 0) * _T
               + lax.broadcasted_iota(jnp.int32, (_NT1, _T), 1)
               ).astype(jnp.float32)
        pright = pos >= s2
        real = (pos < c1) | (pright & (pos < s2 + c2))
        dst = jnp.zeros_like(d)
        t = jnp.zeros((1, 1), jnp.float32)
        ts, es = [], []
        for b in range(4):
            want_right = (b // 2) == 1
            want_d = (b % 2) == 0
            m = (real & (pright == want_right)
                 & ((d > 0.5) == want_d)).astype(jnp.float32)
            rank, cnt = _tri_ranks(m, U, Ls)
            ts.append(t)
            es.append(t + cnt)
            dst = dst + m * (t + rank)
            t = _roundup_t(t + cnt)
        m_tr = 1.0 - real.astype(jnp.float32)
        rank_tr, _ = _tri_ranks(m_tr, U, Ls)
        dst = dst + m_tr * (t + rank_tr)
        dst_ref[...] = dst.astype(jnp.int32)
        meta2_ref[...] = jnp.concatenate(ts + es, axis=1)
        jt = (lax.broadcasted_iota(jnp.int32, (1, _NT2), 1)
              .astype(jnp.float32) * float(_T))
        nm2_ref[...] = 3 + sum(
            (jt >= ts[g]).astype(jnp.int32) for g in (1, 2, 3))


def _run_l1(nm1, rows1, W1, b1r, W2, b2r, W3, b3r, meta1):
    return pl.pallas_call(
        _l1_kernel,
        grid_spec=pltpu.PrefetchScalarGridSpec(
            num_scalar_prefetch=1,
            grid=(_NT1,),
            in_specs=[pl.BlockSpec((_T, _F), lambda j, nm: (j, 0))]
            + _wspecs(lambda j, nm: nm[j])
            + [pl.BlockSpec(memory_space=pltpu.SMEM)],
            out_specs=[
                pl.BlockSpec((_NT1, _T), lambda j, nm: (0, 0)),
                pl.BlockSpec((1, 8), lambda j, nm: (0, 0)),
                pl.BlockSpec((1, _NT2), lambda j, nm: (0, 0)),
            ],
            scratch_shapes=[pltpu.VMEM((_NT1, _T, 1), jnp.float32)],
        ),
        out_shape=[
            jax.ShapeDtypeStruct((_NT1, _T), jnp.int32),
            jax.ShapeDtypeStruct((1, 8), jnp.float32),
            jax.ShapeDtypeStruct((1, _NT2), jnp.int32),
        ],
        compiler_params=pltpu.CompilerParams(
            dimension_semantics=("arbitrary",)),
    )(nm1, rows1, W1, b1r, W2, b2r, W3, b3r, meta1)


def _l2_kernel(nm_ref, x_ref, ids_ref, w1_ref, b1_ref, w2_ref, b2_ref,
               w3_ref, b3_ref, meta_ref, lb_ref, vals_ref, tgt_ref):
    j = pl.program_id(0)
    dd = _mlp_tile(x_ref[...], w1_ref, b1_ref, w2_ref, b2_ref, w3_ref,
                   b3_ref)
    node = nm_ref[j]
    e_g = meta_ref[0, 4 + (node - 3)]
    posi = (j * _T) + lax.broadcasted_iota(jnp.int32, (_T, 1), 0)
    real = posi.astype(jnp.float32) < e_g
    tgt_ref[...] = jnp.where(real, ids_ref[:, 0:1], _N + posi)[None]

    leaf = 2.0 * node.astype(jnp.float32) + 2.0 - dd - 7.0
    out = jnp.zeros_like(dd)
    for k in range(8):
        out = jnp.where(leaf == float(k), lb_ref[k], out)
    vals_ref[...] = jnp.broadcast_to(out, (_T, 128))[None]


def _run_l2(nm2, rows2, ids2, W1, b1r, W2, b2r, W3, b3r, meta2, leaf_best):
    return pl.pallas_call(
        _l2_kernel,
        grid_spec=pltpu.PrefetchScalarGridSpec(
            num_scalar_prefetch=1,
            grid=(_NT2,),
            in_specs=[pl.BlockSpec((_T, _F), lambda j, nm: (j, 0)),
                      pl.BlockSpec((_T, 128), lambda j, nm: (j, 0))]
            + _wspecs(lambda j, nm: nm[j])
            + [pl.BlockSpec(memory_space=pltpu.SMEM),
               pl.BlockSpec(memory_space=pltpu.SMEM)],
            out_specs=[
                pl.BlockSpec((1, _T, 128), lambda j, nm: (j, 0, 0)),
                pl.BlockSpec((1, _T, 1), lambda j, nm: (j, 0, 0)),
            ],
        ),
        out_shape=[
            jax.ShapeDtypeStruct((_NT2, _T, 128), jnp.float32),
            jax.ShapeDtypeStruct((_NT2, _T, 1), jnp.int32),
        ],
        compiler_params=pltpu.CompilerParams(
            dimension_semantics=("arbitrary",)),
    )(nm2, rows2, ids2, W1, b1r, W2, b2r, W3, b3r, meta2, leaf_best)


_NC = 2
_NS = 16
_NWORK = _NC * _NS
_CH = 128


def _sc_mesh():
    return plsc.VectorSubcoreMesh(core_axis_name="c", subcore_axis_name="s",
                                  num_cores=_NC, num_subcores=_NS)


def _sc_permute(src, ids, dst, m_out):
    m_in = src.shape[0]
    nch = m_in // _CH
    nloop = (nch + _NWORK - 1) // _NWORK
    idx = dst.reshape(nch, _CH)
    gen_ids = ids is None

    @functools.partial(
        pl.kernel,
        out_type=(jax.ShapeDtypeStruct((m_out, _F), jnp.float32),
                  jax.ShapeDtypeStruct((m_out, 128), jnp.int32)),
        mesh=_sc_mesh(),
        scratch_types=(
            [pltpu.VMEM((_CH,), jnp.int32) for _ in range(nloop)]
            + [pltpu.VMEM((_CH, _F), jnp.float32) for _ in range(nloop)]
            + [pltpu.VMEM((_CH, 128), jnp.int32) for _ in range(nloop)]
            + [pltpu.SemaphoreType.DMA]
        ),
    )
    def k(*args):
        if gen_ids:
            src_hbm, idx_hbm = args[0], args[1]
            ids_hbm = None
            rest = args[2:]
        else:
            src_hbm, ids_hbm, idx_hbm = args[0], args[1], args[2]
            rest = args[3:]
        out_hbm, ido_hbm = rest[0], rest[1]
        scr = rest[2:]
        idx_vs = scr[:nloop]
        rows_vs = scr[nloop:2 * nloop]
        ids_vs = scr[2 * nloop:3 * nloop]
        sem = scr[3 * nloop]
        wid = lax.axis_index("s") * _NC + lax.axis_index("c")

        def load_fire(t, cid):
            base = cid * _CH
            pltpu.sync_copy(src_hbm.at[pl.ds(base, _CH)], rows_vs[t])
            if gen_ids:
                for r in range(_CH):
                    ids_vs[t][r, pl.ds(0, 16)] = (
                        jnp.zeros((16,), jnp.int32) + (base + r))
            else:
                pltpu.sync_copy(ids_hbm.at[pl.ds(base, _CH)], ids_vs[t])
            pltpu.sync_copy(idx_hbm.at[cid], idx_vs[t])
            c1 = pltpu.async_copy(rows_vs[t], out_hbm.at[idx_vs[t]], sem)
            c2 = pltpu.async_copy(ids_vs[t], ido_hbm.at[idx_vs[t]], sem)
            return c1, c2

        c01, c02 = load_fire(0, wid)
        for t in range(1, nloop):
            @pl.when(wid + _NWORK * t < nch)
            def _():
                c1, c2 = load_fire(t, wid + _NWORK * t)
                c1.wait()
                c2.wait()
        c01.wait()
        c02.wait()

    if gen_ids:
        return k(src, idx)
    return k(src, ids, idx)


def _sc_scatter_out(vals, tgt):
    m = vals.shape[0]
    nch = m // _CH
    nloop = (nch + _NWORK - 1) // _NWORK
    idx = tgt.reshape(nch, _CH)

    @functools.partial(
        pl.kernel,
        out_type=jax.ShapeDtypeStruct((_NPAD, 128), jnp.float32),
        mesh=_sc_mesh(),
        scratch_types=(
            [pltpu.VMEM((_CH,), jnp.int32) for _ in range(nloop)]
            + [pltpu.VMEM((_CH, 128), jnp.float32) for _ in range(nloop)]
            + [pltpu.SemaphoreType.DMA]
        ),
    )
    def k(vals_hbm, idx_hbm, out_hbm, *scr):
        idx_vs = scr[:nloop]
        vals_vs = scr[nloop:2 * nloop]
        sem = scr[2 * nloop]
        wid = lax.axis_index("s") * _NC + lax.axis_index("c")

        def load_fire(t, cid):
            pltpu.sync_copy(vals_hbm.at[pl.ds(cid * _CH, _CH)], vals_vs[t])
            pltpu.sync_copy(idx_hbm.at[cid], idx_vs[t])
            return pltpu.async_copy(vals_vs[t], out_hbm.at[idx_vs[t]], sem)

        c0 = load_fire(0, wid)
        for t in range(1, nloop):
            @pl.when(wid + _NWORK * t < nch)
            def _():
                load_fire(t, wid + _NWORK * t).wait()
        c0.wait()

    return k(vals, idx)


def kernel(x, W1, b1, W2, b2, W3, b3, leaf_best):
    assert x.shape == (_N, _F) and W1.shape == (_NODES, _F, _H)
    b1r = b1[:, None, :]
    b2r = b2[:, None, :]
    b3r = b3[:, None, :]
    weights = (W1, b1r, W2, b2r, W3, b3r)

    dst1, meta1, nm1 = _run_l0(x, *weights)
    rows1, ids1 = _sc_permute(x, None, dst1.reshape(_N), _M1)

    dst2, meta2, nm2 = _run_l1(nm1.reshape(_NT1), rows1, *weights, meta1)
    rows2, ids2 = _sc_permute(rows1, ids1, dst2.reshape(_M1), _M2)

    vals, tgt = _run_l2(nm2.reshape(_NT2), rows2, ids2, *weights, meta2,
                        leaf_best)
    out_pad = _sc_scatter_out(vals.reshape(_M2, 128), tgt.reshape(_M2))
    return out_pad[:_N, 0]


# speedup vs baseline: 1.0865x; 1.0112x over previous
import functools

import jax
import jax.numpy as jnp
from jax import lax
from jax.experimental import pallas as pl
from jax.experimental.pallas import tpu as pltpu
from jax.experimental.pallas import tpu_sc as plsc

_N = 4096
_F = 256
_H = 1024
_NODES = 7
_T = 512
_T0 = 512
_NT0 = _N // _T0
_M1 = _N + _T
_NT1 = _M1 // _T
_M2 = _N + 4 * _T
_NT2 = _M2 // _T
_NPAD = _N + _M2

_HIGH = jax.lax.Precision.HIGHEST


def _mlp_tile(x, node, w1_ref, b1_ref, w2_ref, b2_ref, w3_ref, b3_ref):
    b1 = b1_ref[pl.ds(node, 1), :]
    b2 = b2_ref[pl.ds(node, 1), :]
    w3 = w3_ref[pl.ds(node, 1)][0]
    b3 = b3_ref[pl.ds(node, 1), :]
    h = jnp.tanh(jnp.dot(x, w1_ref[0], preferred_element_type=jnp.float32)
                 + b1)
    h = jnp.tanh(jnp.dot(h, w2_ref[0], preferred_element_type=jnp.float32)
                 + b2)
    logits = jnp.dot(h, w3, preferred_element_type=jnp.float32) + b3
    return (logits[:, 0:1] >= logits[:, 1:2]).astype(jnp.float32)


def _tri_ranks(m, U, Ls):
    r = m.shape[0]
    c = jnp.dot(m, U, precision=_HIGH, preferred_element_type=jnp.float32)
    rowtot = c[:, -1:]
    rowoff = jnp.dot(Ls, rowtot, precision=_HIGH,
                     preferred_element_type=jnp.float32)
    rank = c + rowoff - 1.0
    cnt = rowoff[r - 1:r, :] + rowtot[r - 1:r, :]
    return rank, cnt


def _tris(rows, cols):
    ri = lax.broadcasted_iota(jnp.int32, (cols, cols), 0)
    ci = lax.broadcasted_iota(jnp.int32, (cols, cols), 1)
    U = (ri <= ci).astype(jnp.float32)
    ri2 = lax.broadcasted_iota(jnp.int32, (rows, rows), 0)
    ci2 = lax.broadcasted_iota(jnp.int32, (rows, rows), 1)
    Ls = (ri2 > ci2).astype(jnp.float32)
    return U, Ls


def _roundup_t(v):
    return jnp.floor((v + float(_T - 1)) / float(_T)) * float(_T)


def _wspecs(idx_fn):
    return [
        pl.BlockSpec((1, _F, _H), lambda j, nm: (idx_fn(j, nm), 0, 0)),
        pl.BlockSpec((_NODES, _H), lambda j, nm: (0, 0)),
        pl.BlockSpec((1, _H, _H), lambda j, nm: (idx_fn(j, nm), 0, 0)),
        pl.BlockSpec((_NODES, _H), lambda j, nm: (0, 0)),
        pl.BlockSpec((_NODES, _H, 2), lambda j, nm: (0, 0, 0)),
        pl.BlockSpec((_NODES, 2), lambda j, nm: (0, 0)),
    ]


def _l0_kernel(nm_ref, x_ref, w1_ref, b1_ref, w2_ref, b2_ref, w3_ref,
               b3_ref, dst_ref, meta_ref, nm1_ref, dscr):
    del nm_ref
    j = pl.program_id(0)
    cmp = _mlp_tile(x_ref[...], 0, w1_ref, b1_ref, w2_ref, b2_ref, w3_ref,
                    b3_ref)
    dscr[pl.ds(j, 1)] = cmp[None]

    @pl.when(j == _NT0 - 1)
    def _finish():
        d = dscr[:, :, 0]
        U, Ls = _tris(_NT0, _T0)
        rank_l, cnt0 = _tri_ranks(d, U, Ls)
        rank_r, cnt1 = _tri_ranks(1.0 - d, U, Ls)
        s2 = _roundup_t(cnt0)
        dst = d * rank_l + (1.0 - d) * (s2 + rank_r)
        dst_ref[...] = dst.astype(jnp.int32)
        meta_ref[...] = jnp.concatenate(
            [s2, cnt0, cnt1, jnp.zeros((1, 5), jnp.float32)], axis=1)
        jt = (lax.broadcasted_iota(jnp.int32, (1, _NT1), 1)
              .astype(jnp.float32) * float(_T))
        nm1_ref[...] = 1 + (jt >= s2).astype(jnp.int32)


def _run_l0(xarr, *weights):
    nm0 = jnp.zeros((1,), jnp.int32)
    return pl.pallas_call(
        _l0_kernel,
        grid_spec=pltpu.PrefetchScalarGridSpec(
            num_scalar_prefetch=1,
            grid=(_NT0,),
            in_specs=[pl.BlockSpec((_T0, _F), lambda j, nm: (j, 0))]
            + _wspecs(lambda j, nm: 0),
            out_specs=[
                pl.BlockSpec((_NT0, _T0), lambda j, nm: (0, 0)),
                pl.BlockSpec((1, 8), lambda j, nm: (0, 0)),
                pl.BlockSpec((1, _NT1), lambda j, nm: (0, 0)),
            ],
            scratch_shapes=[pltpu.VMEM((_NT0, _T0, 1), jnp.float32)],
        ),
        out_shape=[
            jax.ShapeDtypeStruct((_NT0, _T0), jnp.int32),
            jax.ShapeDtypeStruct((1, 8), jnp.float32),
            jax.ShapeDtypeStruct((1, _NT1), jnp.int32),
        ],
        compiler_params=pltpu.CompilerParams(
            dimension_semantics=("arbitrary",)),
    )(nm0, xarr, *weights)


def _l1_kernel(nm_ref, x_ref, w1_ref, b1_ref, w2_ref, b2_ref, w3_ref,
               b3_ref, meta1_ref, dst_ref, meta2_ref, nm2_ref, dscr):
    j = pl.program_id(0)
    cmp = _mlp_tile(x_ref[...], nm_ref[j], w1_ref, b1_ref, w2_ref, b2_ref,
                    w3_ref, b3_ref)
    dscr[pl.ds(j, 1)] = cmp[None]

    @pl.when(j == _NT1 - 1)
    def _finish():
        d = dscr[:, :, 0]
        s2 = meta1_ref[0, 0]
        c1 = meta1_ref[0, 1]
        c2 = meta1_ref[0, 2]
        U, Ls = _tris(_NT1, _T)
        pos = (lax.broadcasted_iota(jnp.int32, (_NT1, _T), 0) * _T
               + lax.broadcasted_iota(jnp.int32, (_NT1, _T), 1)
               ).astype(jnp.float32)
        pright = pos >= s2
        real = (pos < c1) | (pright & (pos < s2 + c2))
        dst = jnp.zeros_like(d)
        t = jnp.zeros((1, 1), jnp.float32)
        ts, es = [], []
        for b in range(4):
            want_right = (b // 2) == 1
            want_d = (b % 2) == 0
            m = (real & (pright == want_right)
                 & ((d > 0.5) == want_d)).astype(jnp.float32)
            rank, cnt = _tri_ranks(m, U, Ls)
            ts.append(t)
            es.append(t + cnt)
            dst = dst + m * (t + rank)
            t = _roundup_t(t + cnt)
        m_tr = 1.0 - real.astype(jnp.float32)
        rank_tr, _ = _tri_ranks(m_tr, U, Ls)
        dst = dst + m_tr * (t + rank_tr)
        dst_ref[...] = dst.astype(jnp.int32)
        meta2_ref[...] = jnp.concatenate(ts + es, axis=1)
        jt = (lax.broadcasted_iota(jnp.int32, (1, _NT2), 1)
              .astype(jnp.float32) * float(_T))
        nm2_ref[...] = 3 + sum(
            (jt >= ts[g]).astype(jnp.int32) for g in (1, 2, 3))


def _run_l1(nm1, rows1, W1, b1r, W2, b2r, W3, b3r, meta1):
    return pl.pallas_call(
        _l1_kernel,
        grid_spec=pltpu.PrefetchScalarGridSpec(
            num_scalar_prefetch=1,
            grid=(_NT1,),
            in_specs=[pl.BlockSpec((_T, _F), lambda j, nm: (j, 0))]
            + _wspecs(lambda j, nm: nm[j])
            + [pl.BlockSpec(memory_space=pltpu.SMEM)],
            out_specs=[
                pl.BlockSpec((_NT1, _T), lambda j, nm: (0, 0)),
                pl.BlockSpec((1, 8), lambda j, nm: (0, 0)),
                pl.BlockSpec((1, _NT2), lambda j, nm: (0, 0)),
            ],
            scratch_shapes=[pltpu.VMEM((_NT1, _T, 1), jnp.float32)],
        ),
        out_shape=[
            jax.ShapeDtypeStruct((_NT1, _T), jnp.int32),
            jax.ShapeDtypeStruct((1, 8), jnp.float32),
            jax.ShapeDtypeStruct((1, _NT2), jnp.int32),
        ],
        compiler_params=pltpu.CompilerParams(
            dimension_semantics=("arbitrary",)),
    )(nm1, rows1, W1, b1r, W2, b2r, W3, b3r, meta1)


def _l2_kernel(nm_ref, x_ref, ids_ref, w1_ref, b1_ref, w2_ref, b2_ref,
               w3_ref, b3_ref, meta_ref, lb_ref, vals_ref, tgt_ref):
    j = pl.program_id(0)
    node = nm_ref[j]
    dd = _mlp_tile(x_ref[...], node, w1_ref, b1_ref, w2_ref, b2_ref,
                   w3_ref, b3_ref)
    e_g = meta_ref[0, 4 + (node - 3)]
    posi = (j * _T) + lax.broadcasted_iota(jnp.int32, (_T, 1), 0)
    real = posi.astype(jnp.float32) < e_g
    tgt_ref[...] = jnp.where(real, ids_ref[:, 0:1], _N + posi)[None]

    leaf = 2.0 * node.astype(jnp.float32) + 2.0 - dd - 7.0
    out = jnp.zeros_like(dd)
    for k in range(8):
        out = jnp.where(leaf == float(k), lb_ref[k], out)
    vals_ref[...] = jnp.broadcast_to(out, (_T, 128))[None]


def _run_l2(nm2, rows2, ids2, W1, b1r, W2, b2r, W3, b3r, meta2, leaf_best):
    return pl.pallas_call(
        _l2_kernel,
        grid_spec=pltpu.PrefetchScalarGridSpec(
            num_scalar_prefetch=1,
            grid=(_NT2,),
            in_specs=[pl.BlockSpec((_T, _F), lambda j, nm: (j, 0)),
                      pl.BlockSpec((_T, 128), lambda j, nm: (j, 0))]
            + _wspecs(lambda j, nm: nm[j])
            + [pl.BlockSpec(memory_space=pltpu.SMEM),
               pl.BlockSpec(memory_space=pltpu.SMEM)],
            out_specs=[
                pl.BlockSpec((1, _T, 128), lambda j, nm: (j, 0, 0)),
                pl.BlockSpec((1, _T, 1), lambda j, nm: (j, 0, 0)),
            ],
        ),
        out_shape=[
            jax.ShapeDtypeStruct((_NT2, _T, 128), jnp.float32),
            jax.ShapeDtypeStruct((_NT2, _T, 1), jnp.int32),
        ],
        compiler_params=pltpu.CompilerParams(
            dimension_semantics=("arbitrary",)),
    )(nm2, rows2, ids2, W1, b1r, W2, b2r, W3, b3r, meta2, leaf_best)


_NC = 2
_NS = 16
_NWORK = _NC * _NS
_CH = 128


def _sc_mesh():
    return plsc.VectorSubcoreMesh(core_axis_name="c", subcore_axis_name="s",
                                  num_cores=_NC, num_subcores=_NS)


def _sc_permute(src, ids, dst, m_out):
    m_in = src.shape[0]
    nch = m_in // _CH
    nloop = (nch + _NWORK - 1) // _NWORK
    idx = dst.reshape(nch, _CH)
    gen_ids = ids is None

    @functools.partial(
        pl.kernel,
        out_type=(jax.ShapeDtypeStruct((m_out, _F), jnp.float32),
                  jax.ShapeDtypeStruct((m_out, 128), jnp.int32)),
        mesh=_sc_mesh(),
        scratch_types=(
            [pltpu.VMEM((_CH,), jnp.int32) for _ in range(nloop)]
            + [pltpu.VMEM((_CH, _F), jnp.float32) for _ in range(nloop)]
            + [pltpu.VMEM((_CH, 128), jnp.int32) for _ in range(nloop)]
            + [pltpu.SemaphoreType.DMA]
        ),
    )
    def k(*args):
        if gen_ids:
            src_hbm, idx_hbm = args[0], args[1]
            ids_hbm = None
            rest = args[2:]
        else:
            src_hbm, ids_hbm, idx_hbm = args[0], args[1], args[2]
            rest = args[3:]
        out_hbm, ido_hbm = rest[0], rest[1]
        scr = rest[2:]
        idx_vs = scr[:nloop]
        rows_vs = scr[nloop:2 * nloop]
        ids_vs = scr[2 * nloop:3 * nloop]
        sem = scr[3 * nloop]
        wid = lax.axis_index("s") * _NC + lax.axis_index("c")

        def load_fire(t, cid):
            base = cid * _CH
            pltpu.sync_copy(src_hbm.at[pl.ds(base, _CH)], rows_vs[t])
            if gen_ids:
                for r in range(_CH):
                    ids_vs[t][r, pl.ds(0, 16)] = (
                        jnp.zeros((16,), jnp.int32) + (base + r))
            else:
                pltpu.sync_copy(ids_hbm.at[pl.ds(base, _CH)], ids_vs[t])
            pltpu.sync_copy(idx_hbm.at[cid], idx_vs[t])
            c1 = pltpu.async_copy(rows_vs[t], out_hbm.at[idx_vs[t]], sem)
            c2 = pltpu.async_copy(ids_vs[t], ido_hbm.at[idx_vs[t]], sem)
            return c1, c2

        c01, c02 = load_fire(0, wid)
        for t in range(1, nloop):
            @pl.when(wid + _NWORK * t < nch)
            def _():
                c1, c2 = load_fire(t, wid + _NWORK * t)
                c1.wait()
                c2.wait()
        c01.wait()
        c02.wait()

    if gen_ids:
        return k(src, idx)
    return k(src, ids, idx)


def _sc_scatter_out(vals, tgt):
    m = vals.shape[0]
    nch = m // _CH
    nloop = (nch + _NWORK - 1) // _NWORK
    idx = tgt.reshape(nch, _CH)

    @functools.partial(
        pl.kernel,
        out_type=jax.ShapeDtypeStruct((_NPAD, 128), jnp.float32),
        mesh=_sc_mesh(),
        scratch_types=(
            [pltpu.VMEM((_CH,), jnp.int32) for _ in range(nloop)]
            + [pltpu.VMEM((_CH, 128), jnp.float32) for _ in range(nloop)]
            + [pltpu.SemaphoreType.DMA]
        ),
    )
    def k(vals_hbm, idx_hbm, out_hbm, *scr):
        idx_vs = scr[:nloop]
        vals_vs = scr[nloop:2 * nloop]
        sem = scr[2 * nloop]
        wid = lax.axis_index("s") * _NC + lax.axis_index("c")

        def load_fire(t, cid):
            pltpu.sync_copy(vals_hbm.at[pl.ds(cid * _CH, _CH)], vals_vs[t])
            pltpu.sync_copy(idx_hbm.at[cid], idx_vs[t])
            return pltpu.async_copy(vals_vs[t], out_hbm.at[idx_vs[t]], sem)

        c0 = load_fire(0, wid)
        for t in range(1, nloop):
            @pl.when(wid + _NWORK * t < nch)
            def _():
                load_fire(t, wid + _NWORK * t).wait()
        c0.wait()

    return k(vals, idx)


def kernel(x, W1, b1, W2, b2, W3, b3, leaf_best):
    assert x.shape == (_N, _F) and W1.shape == (_NODES, _F, _H)
    weights = (W1, b1, W2, b2, W3, b3)

    dst1, meta1, nm1 = _run_l0(x, *weights)
    rows1, ids1 = _sc_permute(x, None, dst1.reshape(_N), _M1)

    dst2, meta2, nm2 = _run_l1(nm1.reshape(_NT1), rows1, *weights, meta1)
    rows2, ids2 = _sc_permute(rows1, ids1, dst2.reshape(_M1), _M2)

    vals, tgt = _run_l2(nm2.reshape(_NT2), rows2, ids2, *weights, meta2,
                        leaf_best)
    out_pad = _sc_scatter_out(vals.reshape(_M2, 128), tgt.reshape(_M2))
    return out_pad[:_N, 0]


# speedup vs baseline: 1.1064x; 1.0183x over previous
import functools

import jax
import jax.numpy as jnp
from jax import lax
from jax.experimental import pallas as pl
from jax.experimental.pallas import tpu as pltpu
from jax.experimental.pallas import tpu_sc as plsc

_N = 4096
_F = 256
_H = 1024
_NODES = 7
_T = 512
_T0 = 512
_NT0 = _N // _T0
_M1 = _N + _T
_NT1 = _M1 // _T
_M2 = _N + 4 * _T
_NT2 = _M2 // _T
_NPAD = _N + _M2

_HIGH = jax.lax.Precision.HIGHEST


def _mlp_tile(x, node, w1_ref, b1_ref, w2_ref, b2_ref, w3_ref, b3_ref):
    b1 = b1_ref[pl.ds(node, 1), :]
    b2 = b2_ref[pl.ds(node, 1), :]
    w3 = w3_ref[pl.ds(node, 1)][0]
    b3 = b3_ref[pl.ds(node, 1), :]
    h = jnp.tanh(jnp.dot(x, w1_ref[0], preferred_element_type=jnp.float32)
                 + b1)
    h = jnp.tanh(jnp.dot(h, w2_ref[0], preferred_element_type=jnp.float32)
                 + b2)
    logits = jnp.dot(h, w3, preferred_element_type=jnp.float32) + b3
    return (logits[:, 0:1] >= logits[:, 1:2]).astype(jnp.float32)


def _tri_ranks(m, U, Ls):
    r = m.shape[0]
    c = jnp.dot(m, U, precision=_HIGH, preferred_element_type=jnp.float32)
    rowtot = c[:, -1:]
    rowoff = jnp.dot(Ls, rowtot, precision=_HIGH,
                     preferred_element_type=jnp.float32)
    rank = c + rowoff - 1.0
    cnt = rowoff[r - 1:r, :] + rowtot[r - 1:r, :]
    return rank, cnt


def _tris(rows, cols):
    ri = lax.broadcasted_iota(jnp.int32, (cols, cols), 0)
    ci = lax.broadcasted_iota(jnp.int32, (cols, cols), 1)
    U = (ri <= ci).astype(jnp.float32)
    ri2 = lax.broadcasted_iota(jnp.int32, (rows, rows), 0)
    ci2 = lax.broadcasted_iota(jnp.int32, (rows, rows), 1)
    Ls = (ri2 > ci2).astype(jnp.float32)
    return U, Ls


def _roundup_t(v):
    return jnp.floor((v + float(_T - 1)) / float(_T)) * float(_T)


def _wspecs(idx_fn):
    return [
        pl.BlockSpec((1, _F, _H), lambda j, nm: (idx_fn(j, nm), 0, 0)),
        pl.BlockSpec((_NODES, _H), lambda j, nm: (0, 0)),
        pl.BlockSpec((1, _H, _H), lambda j, nm: (idx_fn(j, nm), 0, 0)),
        pl.BlockSpec((_NODES, _H), lambda j, nm: (0, 0)),
        pl.BlockSpec((_NODES, _H, 2), lambda j, nm: (0, 0, 0)),
        pl.BlockSpec((_NODES, 2), lambda j, nm: (0, 0)),
    ]


def _l0_kernel(nm_ref, x_ref, w1_ref, b1_ref, w2_ref, b2_ref, w3_ref,
               b3_ref, dst_ref, meta_ref, nm1_ref, dscr):
    del nm_ref
    j = pl.program_id(0)
    cmp = _mlp_tile(x_ref[...], 0, w1_ref, b1_ref, w2_ref, b2_ref, w3_ref,
                    b3_ref)
    dscr[pl.ds(j, 1)] = cmp[None]

    @pl.when(j == _NT0 - 1)
    def _finish():
        d = dscr[:, :, 0]
        U, Ls = _tris(_NT0, _T0)
        rank_l, cnt0 = _tri_ranks(d, U, Ls)
        rank_r, cnt1 = _tri_ranks(1.0 - d, U, Ls)
        s2 = _roundup_t(cnt0)
        dst = d * rank_l + (1.0 - d) * (s2 + rank_r)
        dst_ref[...] = dst.astype(jnp.int32)
        meta_ref[...] = jnp.concatenate(
            [s2, cnt0, cnt1, jnp.zeros((1, 5), jnp.float32)], axis=1)
        jt = (lax.broadcasted_iota(jnp.int32, (1, _NT1), 1)
              .astype(jnp.float32) * float(_T))
        nm1_ref[...] = 1 + (jt >= s2).astype(jnp.int32)


def _run_l0(xarr, *weights):
    nm0 = jnp.zeros((1,), jnp.int32)
    return pl.pallas_call(
        _l0_kernel,
        grid_spec=pltpu.PrefetchScalarGridSpec(
            num_scalar_prefetch=1,
            grid=(_NT0,),
            in_specs=[pl.BlockSpec((_T0, _F), lambda j, nm: (j, 0))]
            + _wspecs(lambda j, nm: 0),
            out_specs=[
                pl.BlockSpec((_NT0, _T0), lambda j, nm: (0, 0)),
                pl.BlockSpec((1, 8), lambda j, nm: (0, 0)),
                pl.BlockSpec((1, _NT1), lambda j, nm: (0, 0)),
            ],
            scratch_shapes=[pltpu.VMEM((_NT0, _T0, 1), jnp.float32)],
        ),
        out_shape=[
            jax.ShapeDtypeStruct((_NT0, _T0), jnp.int32),
            jax.ShapeDtypeStruct((1, 8), jnp.float32),
            jax.ShapeDtypeStruct((1, _NT1), jnp.int32),
        ],
        compiler_params=pltpu.CompilerParams(
            dimension_semantics=("arbitrary",)),
    )(nm0, xarr, *weights)


def _l1_kernel(nm_ref, x_ref, w1_ref, b1_ref, w2_ref, b2_ref, w3_ref,
               b3_ref, meta1_ref, dst_ref, meta2_ref, nm2_ref, dscr):
    j = pl.program_id(0)
    node = nm_ref[j]
    s2s = meta1_ref[0, 0]
    ends = jnp.where(node == 1, meta1_ref[0, 1], s2s + meta1_ref[0, 2])

    @pl.when(jnp.float32(j * _T) < ends)
    def _compute():
        cmp = _mlp_tile(x_ref[...], node, w1_ref, b1_ref, w2_ref, b2_ref,
                        w3_ref, b3_ref)
        dscr[pl.ds(j, 1)] = cmp[None]

    @pl.when(j == _NT1 - 1)
    def _finish():
        d = dscr[:, :, 0]
        s2 = meta1_ref[0, 0]
        c1 = meta1_ref[0, 1]
        c2 = meta1_ref[0, 2]
        U, Ls = _tris(_NT1, _T)
        pos = (lax.broadcasted_iota(jnp.int32, (_NT1, _T), 0) * _T
               + lax.broadcasted_iota(jnp.int32, (_NT1, _T), 1)
               ).astype(jnp.float32)
        pright = pos >= s2
        real = (pos < c1) | (pright & (pos < s2 + c2))
        dst = jnp.zeros_like(d)
        t = jnp.zeros((1, 1), jnp.float32)
        ts, es = [], []
        for b in range(4):
            want_right = (b // 2) == 1
            want_d = (b % 2) == 0
            m = (real & (pright == want_right)
                 & ((d > 0.5) == want_d)).astype(jnp.float32)
            rank, cnt = _tri_ranks(m, U, Ls)
            ts.append(t)
            es.append(t + cnt)
            dst = dst + m * (t + rank)
            t = _roundup_t(t + cnt)
        m_tr = 1.0 - real.astype(jnp.float32)
        rank_tr, _ = _tri_ranks(m_tr, U, Ls)
        dst = dst + m_tr * (t + rank_tr)
        dst_ref[...] = dst.astype(jnp.int32)
        meta2_ref[...] = jnp.concatenate(ts + es, axis=1)
        jt = (lax.broadcasted_iota(jnp.int32, (1, _NT2), 1)
              .astype(jnp.float32) * float(_T))
        nm2_ref[...] = 3 + sum(
            (jt >= ts[g]).astype(jnp.int32) for g in (1, 2, 3))


def _run_l1(nm1, rows1, W1, b1r, W2, b2r, W3, b3r, meta1):
    return pl.pallas_call(
        _l1_kernel,
        grid_spec=pltpu.PrefetchScalarGridSpec(
            num_scalar_prefetch=1,
            grid=(_NT1,),
            in_specs=[pl.BlockSpec((_T, _F), lambda j, nm: (j, 0))]
            + _wspecs(lambda j, nm: nm[j])
            + [pl.BlockSpec(memory_space=pltpu.SMEM)],
            out_specs=[
                pl.BlockSpec((_NT1, _T), lambda j, nm: (0, 0)),
                pl.BlockSpec((1, 8), lambda j, nm: (0, 0)),
                pl.BlockSpec((1, _NT2), lambda j, nm: (0, 0)),
            ],
            scratch_shapes=[pltpu.VMEM((_NT1, _T, 1), jnp.float32)],
        ),
        out_shape=[
            jax.ShapeDtypeStruct((_NT1, _T), jnp.int32),
            jax.ShapeDtypeStruct((1, 8), jnp.float32),
            jax.ShapeDtypeStruct((1, _NT2), jnp.int32),
        ],
        compiler_params=pltpu.CompilerParams(
            dimension_semantics=("arbitrary",)),
    )(nm1, rows1, W1, b1r, W2, b2r, W3, b3r, meta1)


def _l2_kernel(nm_ref, x_ref, ids_ref, w1_ref, b1_ref, w2_ref, b2_ref,
               w3_ref, b3_ref, meta_ref, lb_ref, vals_ref, tgt_ref):
    j = pl.program_id(0)
    node = nm_ref[j]
    e_g = meta_ref[0, 4 + (node - 3)]
    posi = (j * _T) + lax.broadcasted_iota(jnp.int32, (_T, 1), 0)
    real = posi.astype(jnp.float32) < e_g
    tgt_ref[...] = jnp.where(real, ids_ref[:, 0:1], _N + posi)[None]

    @pl.when(jnp.float32(j * _T) < e_g)
    def _compute():
        dd = _mlp_tile(x_ref[...], node, w1_ref, b1_ref, w2_ref, b2_ref,
                       w3_ref, b3_ref)
        leaf = 2.0 * node.astype(jnp.float32) + 2.0 - dd - 7.0
        out = jnp.zeros_like(dd)
        for k in range(8):
            out = jnp.where(leaf == float(k), lb_ref[k], out)
        vals_ref[...] = jnp.broadcast_to(out, (_T, 128))[None]


def _run_l2(nm2, rows2, ids2, W1, b1r, W2, b2r, W3, b3r, meta2, leaf_best):
    return pl.pallas_call(
        _l2_kernel,
        grid_spec=pltpu.PrefetchScalarGridSpec(
            num_scalar_prefetch=1,
            grid=(_NT2,),
            in_specs=[pl.BlockSpec((_T, _F), lambda j, nm: (j, 0)),
                      pl.BlockSpec((_T, 128), lambda j, nm: (j, 0))]
            + _wspecs(lambda j, nm: nm[j])
            + [pl.BlockSpec(memory_space=pltpu.SMEM),
               pl.BlockSpec(memory_space=pltpu.SMEM)],
            out_specs=[
                pl.BlockSpec((1, _T, 128), lambda j, nm: (j, 0, 0)),
                pl.BlockSpec((1, _T, 1), lambda j, nm: (j, 0, 0)),
            ],
        ),
        out_shape=[
            jax.ShapeDtypeStruct((_NT2, _T, 128), jnp.float32),
            jax.ShapeDtypeStruct((_NT2, _T, 1), jnp.int32),
        ],
        compiler_params=pltpu.CompilerParams(
            dimension_semantics=("arbitrary",)),
    )(nm2, rows2, ids2, W1, b1r, W2, b2r, W3, b3r, meta2, leaf_best)


_NC = 2
_NS = 16
_NWORK = _NC * _NS
_CH = 128


def _sc_mesh():
    return plsc.VectorSubcoreMesh(core_axis_name="c", subcore_axis_name="s",
                                  num_cores=_NC, num_subcores=_NS)


def _sc_permute(src, ids, dst, m_out):
    m_in = src.shape[0]
    nch = m_in // _CH
    nloop = (nch + _NWORK - 1) // _NWORK
    idx = dst.reshape(nch, _CH)
    gen_ids = ids is None

    @functools.partial(
        pl.kernel,
        out_type=(jax.ShapeDtypeStruct((m_out, _F), jnp.float32),
                  jax.ShapeDtypeStruct((m_out, 128), jnp.int32)),
        mesh=_sc_mesh(),
        scratch_types=(
            [pltpu.VMEM((_CH,), jnp.int32) for _ in range(nloop)]
            + [pltpu.VMEM((_CH, _F), jnp.float32) for _ in range(nloop)]
            + [pltpu.VMEM((_CH, 128), jnp.int32) for _ in range(nloop)]
            + [pltpu.SemaphoreType.DMA]
        ),
    )
    def k(*args):
        if gen_ids:
            src_hbm, idx_hbm = args[0], args[1]
            ids_hbm = None
            rest = args[2:]
        else:
            src_hbm, ids_hbm, idx_hbm = args[0], args[1], args[2]
            rest = args[3:]
        out_hbm, ido_hbm = rest[0], rest[1]
        scr = rest[2:]
        idx_vs = scr[:nloop]
        rows_vs = scr[nloop:2 * nloop]
        ids_vs = scr[2 * nloop:3 * nloop]
        sem = scr[3 * nloop]
        wid = lax.axis_index("s") * _NC + lax.axis_index("c")

        def load_fire(t, cid):
            base = cid * _CH
            pltpu.sync_copy(src_hbm.at[pl.ds(base, _CH)], rows_vs[t])
            if gen_ids:
                for r in range(_CH):
                    ids_vs[t][r, pl.ds(0, 16)] = (
                        jnp.zeros((16,), jnp.int32) + (base + r))
            else:
                pltpu.sync_copy(ids_hbm.at[pl.ds(base, _CH)], ids_vs[t])
            pltpu.sync_copy(idx_hbm.at[cid], idx_vs[t])
            c1 = pltpu.async_copy(rows_vs[t], out_hbm.at[idx_vs[t]], sem)
            c2 = pltpu.async_copy(ids_vs[t], ido_hbm.at[idx_vs[t]], sem)
            return c1, c2

        c01, c02 = load_fire(0, wid)
        for t in range(1, nloop):
            @pl.when(wid + _NWORK * t < nch)
            def _():
                c1, c2 = load_fire(t, wid + _NWORK * t)
                c1.wait()
                c2.wait()
        c01.wait()
        c02.wait()

    if gen_ids:
        return k(src, idx)
    return k(src, ids, idx)


def _sc_scatter_out(vals, tgt):
    m = vals.shape[0]
    nch = m // _CH
    nloop = (nch + _NWORK - 1) // _NWORK
    idx = tgt.reshape(nch, _CH)

    @functools.partial(
        pl.kernel,
        out_type=jax.ShapeDtypeStruct((_NPAD, 128), jnp.float32),
        mesh=_sc_mesh(),
        scratch_types=(
            [pltpu.VMEM((_CH,), jnp.int32) for _ in range(nloop)]
            + [pltpu.VMEM((_CH, 128), jnp.float32) for _ in range(nloop)]
            + [pltpu.SemaphoreType.DMA]
        ),
    )
    def k(vals_hbm, idx_hbm, out_hbm, *scr):
        idx_vs = scr[:nloop]
        vals_vs = scr[nloop:2 * nloop]
        sem = scr[2 * nloop]
        wid = lax.axis_index("s") * _NC + lax.axis_index("c")

        def load_fire(t, cid):
            pltpu.sync_copy(vals_hbm.at[pl.ds(cid * _CH, _CH)], vals_vs[t])
            pltpu.sync_copy(idx_hbm.at[cid], idx_vs[t])
            return pltpu.async_copy(vals_vs[t], out_hbm.at[idx_vs[t]], sem)

        c0 = load_fire(0, wid)
        for t in range(1, nloop):
            @pl.when(wid + _NWORK * t < nch)
            def _():
                load_fire(t, wid + _NWORK * t).wait()
        c0.wait()

    return k(vals, idx)


def kernel(x, W1, b1, W2, b2, W3, b3, leaf_best):
    assert x.shape == (_N, _F) and W1.shape == (_NODES, _F, _H)
    weights = (W1, b1, W2, b2, W3, b3)

    dst1, meta1, nm1 = _run_l0(x, *weights)
    rows1, ids1 = _sc_permute(x, None, dst1.reshape(_N), _M1)

    dst2, meta2, nm2 = _run_l1(nm1.reshape(_NT1), rows1, *weights, meta1)
    rows2, ids2 = _sc_permute(rows1, ids1, dst2.reshape(_M1), _M2)

    vals, tgt = _run_l2(nm2.reshape(_NT2), rows2, ids2, *weights, meta2,
                        leaf_best)
    out_pad = _sc_scatter_out(vals.reshape(_M2, 128), tgt.reshape(_M2))
    return out_pad[:_N, 0]


# speedup vs baseline: 1.1447x; 1.0346x over previous
import functools

import jax
import jax.numpy as jnp
from jax import lax
from jax.experimental import pallas as pl
from jax.experimental.pallas import tpu as pltpu
from jax.experimental.pallas import tpu_sc as plsc

_N = 4096
_F = 256
_H = 1024
_NODES = 7
_T = 512
_T0 = 512
_NT0 = _N // _T0
_M1 = _N + _T
_NT1 = _M1 // _T
_M2 = _N + 4 * _T
_NT2 = _M2 // _T
_NPAD = _N + _M2

_HIGH = jax.lax.Precision.HIGHEST


def _mlp_tile(x, node, w1_ref, b1_ref, w2_ref, b2_ref, w3_ref, b3_ref):
    b1 = b1_ref[pl.ds(node, 1), :]
    b2 = b2_ref[pl.ds(node, 1), :]
    w3 = w3_ref[pl.ds(node, 1)][0]
    b3 = b3_ref[pl.ds(node, 1), :]
    h = jnp.tanh(jnp.dot(x, w1_ref[0], preferred_element_type=jnp.float32)
                 + b1)
    h = jnp.tanh(jnp.dot(h, w2_ref[0], preferred_element_type=jnp.float32)
                 + b2)
    logits = jnp.dot(h, w3, preferred_element_type=jnp.float32) + b3
    return (logits[:, 0:1] >= logits[:, 1:2]).astype(jnp.float32)


def _tri_ranks(m, U, Ls):
    r = m.shape[0]
    c = jnp.dot(m, U, precision=_HIGH, preferred_element_type=jnp.float32)
    rowtot = c[:, -1:]
    rowoff = jnp.dot(Ls, rowtot, precision=_HIGH,
                     preferred_element_type=jnp.float32)
    rank = c + rowoff - 1.0
    cnt = rowoff[r - 1:r, :] + rowtot[r - 1:r, :]
    return rank, cnt


def _tris(rows, cols):
    ri = lax.broadcasted_iota(jnp.int32, (cols, cols), 0)
    ci = lax.broadcasted_iota(jnp.int32, (cols, cols), 1)
    U = (ri <= ci).astype(jnp.float32)
    ri2 = lax.broadcasted_iota(jnp.int32, (rows, rows), 0)
    ci2 = lax.broadcasted_iota(jnp.int32, (rows, rows), 1)
    Ls = (ri2 > ci2).astype(jnp.float32)
    return U, Ls


def _roundup_t(v):
    return jnp.floor((v + float(_T - 1)) / float(_T)) * float(_T)


def _wspecs(idx_fn):
    return [
        pl.BlockSpec((1, _F, _H), lambda j, nm: (idx_fn(j, nm), 0, 0)),
        pl.BlockSpec((_NODES, _H), lambda j, nm: (0, 0)),
        pl.BlockSpec((1, _H, _H), lambda j, nm: (idx_fn(j, nm), 0, 0)),
        pl.BlockSpec((_NODES, _H), lambda j, nm: (0, 0)),
        pl.BlockSpec((_NODES, _H, 2), lambda j, nm: (0, 0, 0)),
        pl.BlockSpec((_NODES, 2), lambda j, nm: (0, 0)),
    ]


def _l0_kernel(nm_ref, x_ref, w1_ref, b1_ref, w2_ref, b2_ref, w3_ref,
               b3_ref, dst_ref, meta_ref, nm1_ref, dscr):
    del nm_ref
    j = pl.program_id(0)
    cmp = _mlp_tile(x_ref[...], 0, w1_ref, b1_ref, w2_ref, b2_ref, w3_ref,
                    b3_ref)
    dscr[pl.ds(j, 1)] = cmp[None]

    @pl.when(j == _NT0 - 1)
    def _finish():
        d = dscr[:, :, 0]
        U, Ls = _tris(_NT0, _T0)
        rank_l, cnt0 = _tri_ranks(d, U, Ls)
        rank_r, cnt1 = _tri_ranks(1.0 - d, U, Ls)
        s2 = _roundup_t(cnt0)
        dst = d * rank_l + (1.0 - d) * (s2 + rank_r)
        dst_ref[...] = dst.astype(jnp.int32)
        meta_ref[...] = jnp.concatenate(
            [s2, cnt0, cnt1, jnp.zeros((1, 5), jnp.float32)], axis=1)
        jt = (lax.broadcasted_iota(jnp.int32, (1, _NT1), 1)
              .astype(jnp.float32) * float(_T))
        nm1_ref[...] = 1 + (jt >= s2).astype(jnp.int32)


def _run_l0(xarr, *weights):
    nm0 = jnp.zeros((1,), jnp.int32)
    return pl.pallas_call(
        _l0_kernel,
        grid_spec=pltpu.PrefetchScalarGridSpec(
            num_scalar_prefetch=1,
            grid=(_NT0,),
            in_specs=[pl.BlockSpec((_T0, _F), lambda j, nm: (j, 0))]
            + _wspecs(lambda j, nm: 0),
            out_specs=[
                pl.BlockSpec((_NT0, _T0), lambda j, nm: (0, 0)),
                pl.BlockSpec((1, 8), lambda j, nm: (0, 0)),
                pl.BlockSpec((1, _NT1), lambda j, nm: (0, 0)),
            ],
            scratch_shapes=[pltpu.VMEM((_NT0, _T0, 1), jnp.float32)],
        ),
        out_shape=[
            jax.ShapeDtypeStruct((_NT0, _T0), jnp.int32),
            jax.ShapeDtypeStruct((1, 8), jnp.float32),
            jax.ShapeDtypeStruct((1, _NT1), jnp.int32),
        ],
        compiler_params=pltpu.CompilerParams(
            dimension_semantics=("arbitrary",)),
    )(nm0, xarr, *weights)


def _l1_kernel(nm_ref, x_ref, w1_ref, b1_ref, w2_ref, b2_ref, w3_ref,
               b3_ref, meta1_ref, dst_ref, meta2_ref, nm2_ref, dscr):
    j = pl.program_id(0)
    node = nm_ref[j]
    s2s = meta1_ref[0, 0]
    ends = jnp.where(node == 1, meta1_ref[0, 1], s2s + meta1_ref[0, 2])

    @pl.when(jnp.float32(j * _T) < ends)
    def _compute():
        cmp = _mlp_tile(x_ref[...], node, w1_ref, b1_ref, w2_ref, b2_ref,
                        w3_ref, b3_ref)
        dscr[pl.ds(j, 1)] = cmp[None]

    @pl.when(j == _NT1 - 1)
    def _finish():
        d = dscr[:, :, 0]
        s2 = meta1_ref[0, 0]
        c1 = meta1_ref[0, 1]
        c2 = meta1_ref[0, 2]
        U, Ls = _tris(_NT1, _T)
        pos = (lax.broadcasted_iota(jnp.int32, (_NT1, _T), 0) * _T
               + lax.broadcasted_iota(jnp.int32, (_NT1, _T), 1)
               ).astype(jnp.float32)
        pright = pos >= s2
        real = (pos < c1) | (pright & (pos < s2 + c2))
        dst = jnp.zeros_like(d)
        t = jnp.zeros((1, 1), jnp.float32)
        ts, es = [], []
        for b in range(4):
            want_right = (b // 2) == 1
            want_d = (b % 2) == 0
            m = (real & (pright == want_right)
                 & ((d > 0.5) == want_d)).astype(jnp.float32)
            rank, cnt = _tri_ranks(m, U, Ls)
            ts.append(t)
            es.append(t + cnt)
            dst = dst + m * (t + rank)
            t = _roundup_t(t + cnt)
        m_tr = 1.0 - real.astype(jnp.float32)
        rank_tr, _ = _tri_ranks(m_tr, U, Ls)
        dst = dst + m_tr * (t + rank_tr)
        dst_ref[...] = dst.astype(jnp.int32)
        meta2_ref[...] = jnp.concatenate(ts + es, axis=1)
        jt = (lax.broadcasted_iota(jnp.int32, (1, _NT2), 1)
              .astype(jnp.float32) * float(_T))
        nm2_ref[...] = 3 + sum(
            (jt >= ts[g]).astype(jnp.int32) for g in (1, 2, 3))


def _run_l1(nm1, rows1, W1, b1r, W2, b2r, W3, b3r, meta1):
    return pl.pallas_call(
        _l1_kernel,
        grid_spec=pltpu.PrefetchScalarGridSpec(
            num_scalar_prefetch=1,
            grid=(_NT1,),
            in_specs=[pl.BlockSpec((_T, _F), lambda j, nm: (j, 0))]
            + _wspecs(lambda j, nm: nm[j])
            + [pl.BlockSpec(memory_space=pltpu.SMEM)],
            out_specs=[
                pl.BlockSpec((_NT1, _T), lambda j, nm: (0, 0)),
                pl.BlockSpec((1, 8), lambda j, nm: (0, 0)),
                pl.BlockSpec((1, _NT2), lambda j, nm: (0, 0)),
            ],
            scratch_shapes=[pltpu.VMEM((_NT1, _T, 1), jnp.float32)],
        ),
        out_shape=[
            jax.ShapeDtypeStruct((_NT1, _T), jnp.int32),
            jax.ShapeDtypeStruct((1, 8), jnp.float32),
            jax.ShapeDtypeStruct((1, _NT2), jnp.int32),
        ],
        compiler_params=pltpu.CompilerParams(
            dimension_semantics=("arbitrary",)),
    )(nm1, rows1, W1, b1r, W2, b2r, W3, b3r, meta1)


def _l2_kernel(nm_ref, x_ref, ids_ref, w1_ref, b1_ref, w2_ref, b2_ref,
               w3_ref, b3_ref, meta_ref, lb_ref, vals_ref, tgt_ref):
    j = pl.program_id(0)
    node = nm_ref[j]
    e_g = meta_ref[0, 4 + (node - 3)]
    posi = (j * _T) + lax.broadcasted_iota(jnp.int32, (_T, 1), 0)
    real = posi.astype(jnp.float32) < e_g
    tgt_ref[...] = jnp.where(real, ids_ref[:, 0:1], _N + posi)[None]

    @pl.when(jnp.float32(j * _T) < e_g)
    def _compute():
        dd = _mlp_tile(x_ref[...], node, w1_ref, b1_ref, w2_ref, b2_ref,
                       w3_ref, b3_ref)
        leaf = 2.0 * node.astype(jnp.float32) + 2.0 - dd - 7.0
        out = jnp.zeros_like(dd)
        for k in range(8):
            out = jnp.where(leaf == float(k), lb_ref[k], out)
        vals_ref[...] = jnp.broadcast_to(out, (_T, 128))[None]


def _run_l2(nm2, rows2, ids2, W1, b1r, W2, b2r, W3, b3r, meta2, leaf_best):
    return pl.pallas_call(
        _l2_kernel,
        grid_spec=pltpu.PrefetchScalarGridSpec(
            num_scalar_prefetch=1,
            grid=(_NT2,),
            in_specs=[pl.BlockSpec((_T, _F), lambda j, nm: (j, 0)),
                      pl.BlockSpec((_T, 128), lambda j, nm: (j, 0))]
            + _wspecs(lambda j, nm: nm[j])
            + [pl.BlockSpec(memory_space=pltpu.SMEM),
               pl.BlockSpec(memory_space=pltpu.SMEM)],
            out_specs=[
                pl.BlockSpec((1, _T, 128), lambda j, nm: (j, 0, 0)),
                pl.BlockSpec((1, _T, 1), lambda j, nm: (j, 0, 0)),
            ],
        ),
        out_shape=[
            jax.ShapeDtypeStruct((_NT2, _T, 128), jnp.float32),
            jax.ShapeDtypeStruct((_NT2, _T, 1), jnp.int32),
        ],
        compiler_params=pltpu.CompilerParams(
            dimension_semantics=("arbitrary",)),
    )(nm2, rows2, ids2, W1, b1r, W2, b2r, W3, b3r, meta2, leaf_best)


_NC = 2
_NS = 16
_NWORK = _NC * _NS
_CH = 128


def _sc_mesh():
    return plsc.VectorSubcoreMesh(core_axis_name="c", subcore_axis_name="s",
                                  num_cores=_NC, num_subcores=_NS)


def _parts_of(span):
    parts, off = [], 0
    while off < span:
        b = min(128, span - off)
        assert b % 8 == 0
        parts.append((off, b))
        off += b
    return parts


def _sc_permute(src, ids, dst, m_out):
    m_in = src.shape[0]
    span = m_in // _NWORK
    parts = _parts_of(span)
    gen_ids = ids is None

    @functools.partial(
        pl.kernel,
        out_type=(jax.ShapeDtypeStruct((m_out, _F), jnp.float32),
                  jax.ShapeDtypeStruct((m_out, 128), jnp.int32)),
        mesh=_sc_mesh(),
        scratch_types=(
            [pltpu.VMEM((b,), jnp.int32) for _, b in parts]
            + [pltpu.VMEM((span, _F), jnp.float32),
               pltpu.VMEM((span, 128), jnp.int32),
               pltpu.SemaphoreType.DMA]
        ),
    )
    def k(*args):
        if gen_ids:
            src_hbm, idx_hbm = args[0], args[1]
            ids_hbm = None
            rest = args[2:]
        else:
            src_hbm, ids_hbm, idx_hbm = args[0], args[1], args[2]
            rest = args[3:]
        out_hbm, ido_hbm = rest[0], rest[1]
        scr = rest[2:]
        idx_vs = scr[:len(parts)]
        rows_v, ids_v, sem = scr[len(parts)], scr[len(parts) + 1], scr[-1]
        wid = lax.axis_index("s") * _NC + lax.axis_index("c")
        base = wid * span
        pltpu.sync_copy(src_hbm.at[pl.ds(base, span)], rows_v)
        if gen_ids:
            for r in range(span):
                ids_v[r, pl.ds(0, 16)] = (jnp.zeros((16,), jnp.int32)
                                          + (base + r))
        else:
            pltpu.sync_copy(ids_hbm.at[pl.ds(base, span)], ids_v)
        for (off, b), iv in zip(parts, idx_vs):
            pltpu.sync_copy(idx_hbm.at[pl.ds(base + off, b)], iv)
        cs = []
        for (off, b), iv in zip(parts, idx_vs):
            cs.append(pltpu.async_copy(rows_v.at[pl.ds(off, b)],
                                       out_hbm.at[iv], sem))
            cs.append(pltpu.async_copy(ids_v.at[pl.ds(off, b)],
                                       ido_hbm.at[iv], sem))
        for c in cs:
            c.wait()

    if gen_ids:
        return k(src, dst)
    return k(src, ids, dst)


def _sc_scatter_out(vals, tgt):
    m = vals.shape[0]
    span = m // _NWORK
    parts = _parts_of(span)

    @functools.partial(
        pl.kernel,
        out_type=jax.ShapeDtypeStruct((_NPAD, 128), jnp.float32),
        mesh=_sc_mesh(),
        scratch_types=(
            [pltpu.VMEM((b,), jnp.int32) for _, b in parts]
            + [pltpu.VMEM((span, 128), jnp.float32),
               pltpu.SemaphoreType.DMA]
        ),
    )
    def k(vals_hbm, idx_hbm, out_hbm, *scr):
        idx_vs = scr[:len(parts)]
        vals_v, sem = scr[len(parts)], scr[-1]
        wid = lax.axis_index("s") * _NC + lax.axis_index("c")
        base = wid * span
        pltpu.sync_copy(vals_hbm.at[pl.ds(base, span)], vals_v)
        for (off, b), iv in zip(parts, idx_vs):
            pltpu.sync_copy(idx_hbm.at[pl.ds(base + off, b)], iv)
        cs = [pltpu.async_copy(vals_v.at[pl.ds(off, b)], out_hbm.at[iv],
                               sem)
              for (off, b), iv in zip(parts, idx_vs)]
        for c in cs:
            c.wait()

    return k(vals, tgt)


def kernel(x, W1, b1, W2, b2, W3, b3, leaf_best):
    assert x.shape == (_N, _F) and W1.shape == (_NODES, _F, _H)
    weights = (W1, b1, W2, b2, W3, b3)

    dst1, meta1, nm1 = _run_l0(x, *weights)
    rows1, ids1 = _sc_permute(x, None, dst1.reshape(_N), _M1)

    dst2, meta2, nm2 = _run_l1(nm1.reshape(_NT1), rows1, *weights, meta1)
    rows2, ids2 = _sc_permute(rows1, ids1, dst2.reshape(_M1), _M2)

    vals, tgt = _run_l2(nm2.reshape(_NT2), rows2, ids2, *weights, meta2,
                        leaf_best)
    out_pad = _sc_scatter_out(vals.reshape(_M2, 128), tgt.reshape(_M2))
    return out_pad[:_N, 0]


# speedup vs baseline: 1.1607x; 1.0140x over previous
import functools

import jax
import jax.numpy as jnp
from jax import lax
from jax.experimental import pallas as pl
from jax.experimental.pallas import tpu as pltpu
from jax.experimental.pallas import tpu_sc as plsc

_N = 4096
_F = 256
_H = 1024
_NODES = 7
_T = 512
_T0 = 512
_NT0 = _N // _T0
_M1 = _N + _T
_NT1 = _M1 // _T
_M2 = _N + 4 * _T
_NT2 = _M2 // _T
_NPAD = _N + _M2

_HIGH = jax.lax.Precision.HIGHEST


def _mlp_tile(x, node, w1_ref, b1_ref, w2_ref, b2_ref, w3_ref, b3_ref):
    b1 = b1_ref[pl.ds(node, 1), :]
    b2 = b2_ref[pl.ds(node, 1), :]
    w3 = w3_ref[pl.ds(node, 1)][0]
    b3 = b3_ref[pl.ds(node, 1), :]
    h = jnp.tanh(jnp.dot(x, w1_ref[0], preferred_element_type=jnp.float32)
                 + b1)
    h = jnp.tanh(jnp.dot(h, w2_ref[0], preferred_element_type=jnp.float32)
                 + b2)
    logits = jnp.dot(h, w3, preferred_element_type=jnp.float32) + b3
    return (logits[:, 0:1] >= logits[:, 1:2]).astype(jnp.float32)


def _tri_ranks_multi(ms, U, Ls):
    nb = len(ms)
    r = ms[0].shape[0]
    M = jnp.concatenate(ms, axis=0)
    C = jnp.dot(M, U, precision=_HIGH, preferred_element_type=jnp.float32)
    R2 = jnp.concatenate(
        [C[b * r:(b + 1) * r, -1:] for b in range(nb)], axis=1)
    RO = jnp.dot(Ls, R2, precision=_HIGH,
                 preferred_element_type=jnp.float32)
    ranks = [C[b * r:(b + 1) * r, :] + RO[:, b:b + 1] - 1.0
             for b in range(nb)]
    cnts = [RO[r - 1:r, b:b + 1] + R2[r - 1:r, b:b + 1]
            for b in range(nb)]
    return ranks, cnts


def _tris(rows, cols):
    ri = lax.broadcasted_iota(jnp.int32, (cols, cols), 0)
    ci = lax.broadcasted_iota(jnp.int32, (cols, cols), 1)
    U = (ri <= ci).astype(jnp.float32)
    ri2 = lax.broadcasted_iota(jnp.int32, (rows, rows), 0)
    ci2 = lax.broadcasted_iota(jnp.int32, (rows, rows), 1)
    Ls = (ri2 > ci2).astype(jnp.float32)
    return U, Ls


def _roundup_t(v):
    return jnp.floor((v + float(_T - 1)) / float(_T)) * float(_T)


def _wspecs(idx_fn):
    return [
        pl.BlockSpec((1, _F, _H), lambda j, nm: (idx_fn(j, nm), 0, 0)),
        pl.BlockSpec((_NODES, _H), lambda j, nm: (0, 0)),
        pl.BlockSpec((1, _H, _H), lambda j, nm: (idx_fn(j, nm), 0, 0)),
        pl.BlockSpec((_NODES, _H), lambda j, nm: (0, 0)),
        pl.BlockSpec((_NODES, _H, 2), lambda j, nm: (0, 0, 0)),
        pl.BlockSpec((_NODES, 2), lambda j, nm: (0, 0)),
    ]


def _l0_kernel(nm_ref, x_ref, w1_ref, b1_ref, w2_ref, b2_ref, w3_ref,
               b3_ref, dst_ref, meta_ref, nm1_ref, dscr):
    del nm_ref
    j = pl.program_id(0)
    cmp = _mlp_tile(x_ref[...], 0, w1_ref, b1_ref, w2_ref, b2_ref, w3_ref,
                    b3_ref)
    dscr[pl.ds(j, 1)] = cmp[None]

    @pl.when(j == _NT0 - 1)
    def _finish():
        d = dscr[:, :, 0]
        U, Ls = _tris(_NT0, _T0)
        (rank_l, rank_r), (cnt0, cnt1) = _tri_ranks_multi(
            [d, 1.0 - d], U, Ls)
        s2 = _roundup_t(cnt0)
        dst = d * rank_l + (1.0 - d) * (s2 + rank_r)
        dst_ref[...] = dst.astype(jnp.int32)
        meta_ref[...] = jnp.concatenate(
            [s2, cnt0, cnt1, jnp.zeros((1, 5), jnp.float32)], axis=1)
        jt = (lax.broadcasted_iota(jnp.int32, (1, _NT1), 1)
              .astype(jnp.float32) * float(_T))
        nm1_ref[...] = 1 + (jt >= s2).astype(jnp.int32)


def _run_l0(xarr, *weights):
    nm0 = jnp.zeros((1,), jnp.int32)
    return pl.pallas_call(
        _l0_kernel,
        grid_spec=pltpu.PrefetchScalarGridSpec(
            num_scalar_prefetch=1,
            grid=(_NT0,),
            in_specs=[pl.BlockSpec((_T0, _F), lambda j, nm: (j, 0))]
            + _wspecs(lambda j, nm: 0),
            out_specs=[
                pl.BlockSpec((_NT0, _T0), lambda j, nm: (0, 0)),
                pl.BlockSpec((1, 8), lambda j, nm: (0, 0)),
                pl.BlockSpec((1, _NT1), lambda j, nm: (0, 0)),
            ],
            scratch_shapes=[pltpu.VMEM((_NT0, _T0, 1), jnp.float32)],
        ),
        out_shape=[
            jax.ShapeDtypeStruct((_NT0, _T0), jnp.int32),
            jax.ShapeDtypeStruct((1, 8), jnp.float32),
            jax.ShapeDtypeStruct((1, _NT1), jnp.int32),
        ],
        compiler_params=pltpu.CompilerParams(
            dimension_semantics=("arbitrary",)),
    )(nm0, xarr, *weights)


def _l1_kernel(nm_ref, x_ref, w1_ref, b1_ref, w2_ref, b2_ref, w3_ref,
               b3_ref, meta1_ref, dst_ref, meta2_ref, nm2_ref, dscr):
    j = pl.program_id(0)
    node = nm_ref[j]
    s2s = meta1_ref[0, 0]
    ends = jnp.where(node == 1, meta1_ref[0, 1], s2s + meta1_ref[0, 2])

    @pl.when(jnp.float32(j * _T) < ends)
    def _compute():
        cmp = _mlp_tile(x_ref[...], node, w1_ref, b1_ref, w2_ref, b2_ref,
                        w3_ref, b3_ref)
        dscr[pl.ds(j, 1)] = cmp[None]

    @pl.when(j == _NT1 - 1)
    def _finish():
        d = dscr[:, :, 0]
        s2 = meta1_ref[0, 0]
        c1 = meta1_ref[0, 1]
        c2 = meta1_ref[0, 2]
        U, Ls = _tris(_NT1, _T)
        pos = (lax.broadcasted_iota(jnp.int32, (_NT1, _T), 0) * _T
               + lax.broadcasted_iota(jnp.int32, (_NT1, _T), 1)
               ).astype(jnp.float32)
        pright = pos >= s2
        real = (pos < c1) | (pright & (pos < s2 + c2))
        ms = []
        for b in range(4):
            want_right = (b // 2) == 1
            want_d = (b % 2) == 0
            ms.append((real & (pright == want_right)
                       & ((d > 0.5) == want_d)).astype(jnp.float32))
        ms.append(1.0 - real.astype(jnp.float32))
        ranks, cnts = _tri_ranks_multi(ms, U, Ls)
        dst = jnp.zeros_like(d)
        t = jnp.zeros((1, 1), jnp.float32)
        ts, es = [], []
        for b in range(4):
            ts.append(t)
            es.append(t + cnts[b])
            dst = dst + ms[b] * (t + ranks[b])
            t = _roundup_t(t + cnts[b])
        dst = dst + ms[4] * (t + ranks[4])
        dst_ref[...] = dst.astype(jnp.int32)
        meta2_ref[...] = jnp.concatenate(ts + es, axis=1)
        jt = (lax.broadcasted_iota(jnp.int32, (1, _NT2), 1)
              .astype(jnp.float32) * float(_T))
        nm2_ref[...] = 3 + sum(
            (jt >= ts[g]).astype(jnp.int32) for g in (1, 2, 3))


def _run_l1(nm1, rows1, W1, b1r, W2, b2r, W3, b3r, meta1):
    return pl.pallas_call(
        _l1_kernel,
        grid_spec=pltpu.PrefetchScalarGridSpec(
            num_scalar_prefetch=1,
            grid=(_NT1,),
            in_specs=[pl.BlockSpec((_T, _F), lambda j, nm: (j, 0))]
            + _wspecs(lambda j, nm: nm[j])
            + [pl.BlockSpec(memory_space=pltpu.SMEM)],
            out_specs=[
                pl.BlockSpec((_NT1, _T), lambda j, nm: (0, 0)),
                pl.BlockSpec((1, 8), lambda j, nm: (0, 0)),
                pl.BlockSpec((1, _NT2), lambda j, nm: (0, 0)),
            ],
            scratch_shapes=[pltpu.VMEM((_NT1, _T, 1), jnp.float32)],
        ),
        out_shape=[
            jax.ShapeDtypeStruct((_NT1, _T), jnp.int32),
            jax.ShapeDtypeStruct((1, 8), jnp.float32),
            jax.ShapeDtypeStruct((1, _NT2), jnp.int32),
        ],
        compiler_params=pltpu.CompilerParams(
            dimension_semantics=("arbitrary",)),
    )(nm1, rows1, W1, b1r, W2, b2r, W3, b3r, meta1)


def _l2_kernel(nm_ref, x_ref, ids_ref, w1_ref, b1_ref, w2_ref, b2_ref,
               w3_ref, b3_ref, meta_ref, lb_ref, vals_ref, tgt_ref):
    j = pl.program_id(0)
    node = nm_ref[j]
    e_g = meta_ref[0, 4 + (node - 3)]
    posi = (j * _T) + lax.broadcasted_iota(jnp.int32, (_T, 1), 0)
    real = posi.astype(jnp.float32) < e_g
    tgt_ref[...] = jnp.where(real, ids_ref[:, 0:1], _N + posi)[None]

    @pl.when(jnp.float32(j * _T) < e_g)
    def _compute():
        dd = _mlp_tile(x_ref[...], node, w1_ref, b1_ref, w2_ref, b2_ref,
                       w3_ref, b3_ref)
        leaf = 2.0 * node.astype(jnp.float32) + 2.0 - dd - 7.0
        out = jnp.zeros_like(dd)
        for k in range(8):
            out = jnp.where(leaf == float(k), lb_ref[k], out)
        vals_ref[...] = jnp.broadcast_to(out, (_T, 128))[None]


def _run_l2(nm2, rows2, ids2, W1, b1r, W2, b2r, W3, b3r, meta2, leaf_best):
    return pl.pallas_call(
        _l2_kernel,
        grid_spec=pltpu.PrefetchScalarGridSpec(
            num_scalar_prefetch=1,
            grid=(_NT2,),
            in_specs=[pl.BlockSpec((_T, _F), lambda j, nm: (j, 0)),
                      pl.BlockSpec((_T, 128), lambda j, nm: (j, 0))]
            + _wspecs(lambda j, nm: nm[j])
            + [pl.BlockSpec(memory_space=pltpu.SMEM),
               pl.BlockSpec(memory_space=pltpu.SMEM)],
            out_specs=[
                pl.BlockSpec((1, _T, 128), lambda j, nm: (j, 0, 0)),
                pl.BlockSpec((1, _T, 1), lambda j, nm: (j, 0, 0)),
            ],
        ),
        out_shape=[
            jax.ShapeDtypeStruct((_NT2, _T, 128), jnp.float32),
            jax.ShapeDtypeStruct((_NT2, _T, 1), jnp.int32),
        ],
        compiler_params=pltpu.CompilerParams(
            dimension_semantics=("arbitrary",)),
    )(nm2, rows2, ids2, W1, b1r, W2, b2r, W3, b3r, meta2, leaf_best)


_NC = 2
_NS = 16
_NWORK = _NC * _NS
_CH = 128


def _sc_mesh():
    return plsc.VectorSubcoreMesh(core_axis_name="c", subcore_axis_name="s",
                                  num_cores=_NC, num_subcores=_NS)


def _parts_of(span):
    parts, off = [], 0
    while off < span:
        b = min(128, span - off)
        assert b % 8 == 0
        parts.append((off, b))
        off += b
    return parts


def _sc_permute(src, ids, dst, m_out):
    m_in = src.shape[0]
    span = m_in // _NWORK
    parts = _parts_of(span)
    gen_ids = ids is None

    @functools.partial(
        pl.kernel,
        out_type=(jax.ShapeDtypeStruct((m_out, _F), jnp.float32),
                  jax.ShapeDtypeStruct((m_out, 128), jnp.int32)),
        mesh=_sc_mesh(),
        scratch_types=(
            [pltpu.VMEM((b,), jnp.int32) for _, b in parts]
            + [pltpu.VMEM((span, _F), jnp.float32),
               pltpu.VMEM((span, 128), jnp.int32),
               pltpu.SemaphoreType.DMA]
        ),
    )
    def k(*args):
        if gen_ids:
            src_hbm, idx_hbm = args[0], args[1]
            ids_hbm = None
            rest = args[2:]
        else:
            src_hbm, ids_hbm, idx_hbm = args[0], args[1], args[2]
            rest = args[3:]
        out_hbm, ido_hbm = rest[0], rest[1]
        scr = rest[2:]
        idx_vs = scr[:len(parts)]
        rows_v, ids_v, sem = scr[len(parts)], scr[len(parts) + 1], scr[-1]
        wid = lax.axis_index("s") * _NC + lax.axis_index("c")
        base = wid * span
        pltpu.sync_copy(src_hbm.at[pl.ds(base, span)], rows_v)
        if gen_ids:
            for r in range(span):
                ids_v[r, pl.ds(0, 16)] = (jnp.zeros((16,), jnp.int32)
                                          + (base + r))
        else:
            pltpu.sync_copy(ids_hbm.at[pl.ds(base, span)], ids_v)
        for (off, b), iv in zip(parts, idx_vs):
            pltpu.sync_copy(idx_hbm.at[pl.ds(base + off, b)], iv)
        cs = []
        for (off, b), iv in zip(parts, idx_vs):
            cs.append(pltpu.async_copy(rows_v.at[pl.ds(off, b)],
                                       out_hbm.at[iv], sem))
            cs.append(pltpu.async_copy(ids_v.at[pl.ds(off, b)],
                                       ido_hbm.at[iv], sem))
        for c in cs:
            c.wait()

    if gen_ids:
        return k(src, dst)
    return k(src, ids, dst)


def _sc_scatter_out(vals, tgt):
    m = vals.shape[0]
    span = m // _NWORK
    parts = _parts_of(span)

    @functools.partial(
        pl.kernel,
        out_type=jax.ShapeDtypeStruct((_NPAD, 128), jnp.float32),
        mesh=_sc_mesh(),
        scratch_types=(
            [pltpu.VMEM((b,), jnp.int32) for _, b in parts]
            + [pltpu.VMEM((span, 128), jnp.float32),
               pltpu.SemaphoreType.DMA]
        ),
    )
    def k(vals_hbm, idx_hbm, out_hbm, *scr):
        idx_vs = scr[:len(parts)]
        vals_v, sem = scr[len(parts)], scr[-1]
        wid = lax.axis_index("s") * _NC + lax.axis_index("c")
        base = wid * span
        pltpu.sync_copy(vals_hbm.at[pl.ds(base, span)], vals_v)
        for (off, b), iv in zip(parts, idx_vs):
            pltpu.sync_copy(idx_hbm.at[pl.ds(base + off, b)], iv)
        cs = [pltpu.async_copy(vals_v.at[pl.ds(off, b)], out_hbm.at[iv],
                               sem)
              for (off, b), iv in zip(parts, idx_vs)]
        for c in cs:
            c.wait()

    return k(vals, tgt)


def kernel(x, W1, b1, W2, b2, W3, b3, leaf_best):
    assert x.shape == (_N, _F) and W1.shape == (_NODES, _F, _H)
    weights = (W1, b1, W2, b2, W3, b3)

    dst1, meta1, nm1 = _run_l0(x, *weights)
    rows1, ids1 = _sc_permute(x, None, dst1.reshape(_N), _M1)

    dst2, meta2, nm2 = _run_l1(nm1.reshape(_NT1), rows1, *weights, meta1)
    rows2, ids2 = _sc_permute(rows1, ids1, dst2.reshape(_M1), _M2)

    vals, tgt = _run_l2(nm2.reshape(_NT2), rows2, ids2, *weights, meta2,
                        leaf_best)
    out_pad = _sc_scatter_out(vals.reshape(_M2, 128), tgt.reshape(_M2))
    return out_pad[:_N, 0]
